# Initial kernel scaffold; baseline (speedup 1.0000x reference)
import jax
import jax.numpy as jnp
from jax.experimental import pallas as pl


def kernel(points, params):
    raise NotImplementedError("write your pallas kernel here")


# speedup vs baseline: 1.0268x; 1.0268x over previous
import numpy as np
import jax
import jax.numpy as jnp
from jax.experimental import pallas as pl

_EPS = 1e-8
_F32 = jnp.float32


def _pick_br(rows, mult, cap):
    best = mult
    k = 1
    while True:
        cand = mult * (k + 1)
        if cand > cap or cand > rows:
            break
        k += 1
        if rows % cand == 0:
            best = cand
    return best if rows % best == 0 else rows


def _stats_update(s_ref, y):
    c = y.shape[-1]
    row0 = jnp.sum(y, axis=0, keepdims=True)
    row1 = jnp.sum(y * y, axis=0, keepdims=True)
    upd = jnp.concatenate(
        [row0, row1, jnp.zeros((6, c), dtype=y.dtype)], axis=0)

    @pl.when(pl.program_id(0) == 0)
    def _():
        s_ref[...] = jnp.zeros_like(s_ref)

    s_ref[...] += upd


def _mm0_body(x_ref, wt_ref, b_ref, y_ref, s_ref):
    y = jnp.dot(x_ref[...], wt_ref[...],
                preferred_element_type=_F32) + b_ref[...]
    y_ref[...] = y
    _stats_update(s_ref, y)


def _mm1_body(x_ref, a_ref, c_ref, wt_ref, b_ref, y_ref, s_ref):
    h = jnp.maximum(x_ref[...] * a_ref[...] + c_ref[...], 0.0)
    y = jnp.dot(h, wt_ref[...], preferred_element_type=_F32) + b_ref[...]
    y_ref[...] = y
    _stats_update(s_ref, y)


def _mm2_body(x1_ref, a1_ref, c1_ref, x2_ref, a2_ref, c2_ref,
              wt_ref, b_ref, y_ref, s_ref):
    pre = (x1_ref[...] * a1_ref[...] + c1_ref[...]
           + x2_ref[...] * a2_ref[...] + c2_ref[...])
    h = jnp.maximum(pre, 0.0)
    y = jnp.dot(h, wt_ref[...], preferred_element_type=_F32) + b_ref[...]
    y_ref[...] = y
    _stats_update(s_ref, y)


def _pool_body(x_ref, a_ref, c_ref, o_ref, *, relu, mode):
    h = x_ref[...] * a_ref[...] + c_ref[...]
    if relu:
        h = jnp.maximum(h, 0.0)
    if mode == "max":
        o_ref[...] = jnp.max(h, axis=1)
    else:
        o_ref[...] = jnp.sum(h, axis=1)


def _mm_stats(x, wt, b, affine=None, x2=None, affine2=None):
    rows, cin = x.shape
    cout = wt.shape[1]
    pcin = -(-cin // 128) * 128
    pcout = -(-cout // 128) * 128
    nin = 2 if x2 is not None else 1
    cap = max(8, (4 << 20) // (8 * (nin * pcin + pcout)))
    br = _pick_br(rows, 8, min(cap, rows))
    grid = (rows // br,)
    b2 = b.reshape(1, cout)
    out_shape = [jax.ShapeDtypeStruct((rows, cout), _F32),
                 jax.ShapeDtypeStruct((8, cout), _F32)]
    out_specs = [pl.BlockSpec((br, cout), lambda i: (i, 0)),
                 pl.BlockSpec((8, cout), lambda i: (0, 0))]
    w_spec = pl.BlockSpec((cin, cout), lambda i: (0, 0))
    bias_spec = pl.BlockSpec((1, cout), lambda i: (0, 0))
    row_spec = pl.BlockSpec((br, cin), lambda i: (i, 0))
    vec_spec = pl.BlockSpec((1, cin), lambda i: (0, 0))

    if affine is None:
        y, s = pl.pallas_call(
            _mm0_body, grid=grid,
            in_specs=[row_spec, w_spec, bias_spec],
            out_specs=out_specs, out_shape=out_shape,
        )(x, wt, b2)
    elif x2 is None:
        a, c = affine
        y, s = pl.pallas_call(
            _mm1_body, grid=grid,
            in_specs=[row_spec, vec_spec, vec_spec, w_spec, bias_spec],
            out_specs=out_specs, out_shape=out_shape,
        )(x, a.reshape(1, cin), c.reshape(1, cin), wt, b2)
    else:
        a1, c1 = affine
        a2, c2 = affine2
        y, s = pl.pallas_call(
            _mm2_body, grid=grid,
            in_specs=[row_spec, vec_spec, vec_spec,
                      row_spec, vec_spec, vec_spec, w_spec, bias_spec],
            out_specs=out_specs, out_shape=out_shape,
        )(x, a1.reshape(1, cin), c1.reshape(1, cin),
          x2, a2.reshape(1, cin), c2.reshape(1, cin), wt, b2)
    return y, s


def _bn_affine(stats, rows, gamma, beta):
    mean = stats[0] / rows
    var = stats[1] / rows - mean * mean
    a = gamma / jnp.sqrt(var + 1e-5)
    c = beta - mean * a
    return a, c


def _pool(x, a, c, k, relu, mode):
    rows, cin = x.shape
    groups = rows // k
    x3 = x.reshape(groups, k, cin)
    pcin = -(-cin // 128) * 128
    gb = _pick_br(groups, 8, max(8, (1 << 21) // (4 * k * pcin)))
    if groups % gb != 0 or groups < 8:
        gb = groups
    y = pl.pallas_call(
        lambda x_ref, a_ref, c_ref, o_ref: _pool_body(
            x_ref, a_ref, c_ref, o_ref, relu=relu, mode=mode),
        grid=(groups // gb,),
        in_specs=[pl.BlockSpec((gb, k, cin), lambda i: (i, 0, 0)),
                  pl.BlockSpec((1, 1, cin), lambda i: (0, 0, 0)),
                  pl.BlockSpec((1, 1, cin), lambda i: (0, 0, 0))],
        out_specs=pl.BlockSpec((gb, cin), lambda i: (i, 0)),
        out_shape=jax.ShapeDtypeStruct((groups, cin), _F32),
    )(x3, a.reshape(1, 1, cin), c.reshape(1, 1, cin))
    return y


def _cls_body(x_ref, w1_ref, b1_ref, g1_ref, t1_ref,
              w2_ref, b2_ref, g2_ref, t2_ref,
              w3_ref, b3_ref, o_ref):
    def bn_relu(y, g, t):
        mean = jnp.mean(y, axis=0, keepdims=True)
        var = jnp.mean(y * y, axis=0, keepdims=True) - mean * mean
        return jnp.maximum(g * (y - mean) / jnp.sqrt(var + 1e-5) + t, 0.0)

    h = jnp.dot(x_ref[...], w1_ref[...], preferred_element_type=_F32) \
        + b1_ref[...]
    h = bn_relu(h, g1_ref[...], t1_ref[...])
    h = jnp.dot(h, w2_ref[...], preferred_element_type=_F32) + b2_ref[...]
    h = bn_relu(h, g2_ref[...], t2_ref[...])
    z = jnp.dot(h, w3_ref[...], preferred_element_type=_F32) + b3_ref[...]
    m = jnp.max(z, axis=-1, keepdims=True)
    lse = m + jnp.log(jnp.sum(jnp.exp(z - m), axis=-1, keepdims=True))
    o_ref[...] = z - lse


def _classifier(feat, cp):
    b = feat.shape[0]
    args = (feat,
            cp['W1'].T, cp['b1'].reshape(1, -1),
            cp['g1'].reshape(1, -1), cp['bt1'].reshape(1, -1),
            cp['W2'].T, cp['b2'].reshape(1, -1),
            cp['g2'].reshape(1, -1), cp['bt2'].reshape(1, -1),
            cp['W3'].T, cp['b3'].reshape(1, -1))
    return pl.pallas_call(
        _cls_body,
        out_shape=jax.ShapeDtypeStruct((b, cp['W3'].shape[0]), _F32),
    )(*args)


def _xyz2sphere(xyz):
    rho = jnp.sqrt(jnp.sum(xyz ** 2, axis=-1, keepdims=True))
    rho_c = jnp.maximum(rho, _EPS)
    theta = jnp.arccos(jnp.clip(xyz[..., 2:3] / rho_c, -1.0, 1.0)) / np.pi
    phi = jnp.arctan2(xyz[..., 1:2], xyz[..., 0:1]) / (2 * np.pi) + 0.5
    return jnp.concatenate([rho, theta, phi], axis=-1)


def _index_points(points, idx):
    return jax.vmap(lambda p, i: p[i])(points, idx)


def _sq_dist(src, dst):
    return jnp.sum((src[:, :, None, :] - dst[:, None, :, :]) ** 2, axis=-1)


def _fps(xyz, npoint):
    n = xyz.shape[1]

    def single(x):
        def body(i, state):
            dists, far, idxs = state
            idxs = idxs.at[i].set(far)
            d = jnp.sum((x - x[far]) ** 2, axis=-1)
            dists = jnp.minimum(dists, d)
            return dists, jnp.argmax(dists).astype(jnp.int32), idxs

        state = (jnp.full((n,), 1e10, dtype=_F32), jnp.int32(0),
                 jnp.zeros((npoint,), jnp.int32))
        return jax.lax.fori_loop(0, npoint, body, state)[2]

    return jax.vmap(single)(xyz)


def _ball_query(radius, nsample, xyz, new_xyz):
    b, n, _ = xyz.shape
    s = new_xyz.shape[1]
    sqrdists = _sq_dist(new_xyz, xyz)
    gidx = jnp.broadcast_to(jnp.arange(n, dtype=jnp.int32), (b, s, n))
    gidx = jnp.where(sqrdists > radius ** 2, n, gidx)
    gidx = jnp.sort(gidx, axis=-1)[:, :, :nsample]
    first = gidx[:, :, 0:1]
    return jnp.where(gidx == n, first, gidx)


def _knn(k, xyz):
    d = _sq_dist(xyz, xyz)
    return jnp.argsort(d, axis=-1)[:, :, :k]


def _umbrella_surface(center, p, k):
    b, n, _ = center.shape
    idx = _knn(k, center)
    gxyz = _index_points(center, idx)[:, :, 1:]
    gnorm = gxyz - center[:, :, None, :]
    phi = _xyz2sphere(gnorm)[..., 2]
    sidx = jnp.argsort(phi, axis=-1)
    sg = jnp.take_along_axis(
        gnorm, jnp.broadcast_to(sidx[..., None], gnorm.shape), axis=2
    )[:, :, :, None, :]
    sgr = jnp.roll(sg, -1, axis=2)
    g = jnp.concatenate([jnp.zeros_like(sg), sg, sgr], axis=-2)

    e1 = g[..., 1, :] - g[..., 0, :]
    e2 = g[..., 2, :] - g[..., 0, :]
    nor = jnp.cross(e1, e2)
    unit = nor / jnp.maximum(
        jnp.linalg.norm(nor, axis=-1, keepdims=True), _EPS)
    pos_mask = (unit[..., 0:1, 0] > 0).astype(_F32) * 2.0 - 1.0
    unit = unit * pos_mask[..., None]
    gc = jnp.mean(g, axis=-2)
    gpolar = _xyz2sphere(gc)
    gpos = jnp.sum(unit * gc, axis=-1, keepdims=True) / np.sqrt(3.0)
    feat = jnp.concatenate([gc, gpolar, unit, gpos], axis=-1)
    feat = jnp.nan_to_num(feat)

    kk = k - 1
    rows = b * n * kk
    x = feat.reshape(rows, feat.shape[-1])
    y1, s1 = _mm_stats(x, p['W1'].T, p['b1'])
    a1, c1 = _bn_affine(s1, rows, p['g1'], p['bt1'])
    y2, _ = _mm_stats(y1, p['W2'].T, p['b2'], affine=(a1, c1))
    out = _pool(y2, jnp.ones_like(p['b2']), jnp.zeros_like(p['b2']),
                kk, relu=False, mode="sum")
    return out.reshape(b, n, -1)


def _sa_stage(center, normal, feature, p, npoint, radius, nsample,
              pos_channel, group_all):
    b = center.shape[0]
    if group_all:
        new_center = jnp.zeros((b, 1, 3), dtype=center.dtype)
        new_normal = jnp.zeros((b, 1, normal.shape[-1]), dtype=normal.dtype)
        parts = [center[:, None], normal[:, None]]
        if feature is not None:
            parts.append(feature[:, None])
        nf = jnp.concatenate(parts, axis=-1)
        k = center.shape[1]
    else:
        fps_idx = _fps(center, npoint)
        new_center = _index_points(center, fps_idx)
        new_normal = _index_points(normal, fps_idx)
        idx = _ball_query(radius, nsample, center, new_center)
        gpos = _index_points(center, idx) - new_center[:, :, None, :]
        parts = [gpos, _index_points(normal, idx)]
        if feature is not None:
            parts.append(_index_points(feature, idx))
        nf = jnp.concatenate(parts, axis=-1)
        k = nsample

    s = nf.shape[1]
    rows = b * s * k
    cin = nf.shape[-1]
    flat = nf.reshape(rows, cin)
    xp = flat[:, :pos_channel]
    xf = flat[:, pos_channel:]

    yl, sl = _mm_stats(xp, p['Wl0'].T, p['bl0'])
    yf, sf = _mm_stats(xf, p['Wf0'].T, p['bf0'])
    al, cl = _bn_affine(sl, rows, p['gl0'], p['btl0'])
    af, cf = _bn_affine(sf, rows, p['gf0'], p['btf0'])

    y = None
    aff = None
    first = True
    for lay in p['layers']:
        if first:
            y, st = _mm_stats(yl, lay['W'].T, lay['b'],
                              affine=(al, cl), x2=yf, affine2=(af, cf))
            first = False
        else:
            y, st = _mm_stats(y, lay['W'].T, lay['b'], affine=aff)
        aff = _bn_affine(st, rows, lay['g'], lay['bt'])

    f = _pool(y, aff[0], aff[1], k, relu=True, mode="max")
    return new_center, new_normal, f.reshape(b, s, -1)


def kernel(points, params):
    center = jnp.transpose(points[:, :3, :], (0, 2, 1))
    normal = _umbrella_surface(center, params['umb'], 9)
    c, n, f = _sa_stage(center, normal, None, params['sa1'],
                        512, 0.1, 24, 3, False)
    c, n, f = _sa_stage(c, n, f, params['sa2'], 128, 0.2, 24, 3, False)
    c, n, f = _sa_stage(c, n, f, params['sa3'], 32, 0.4, 24, 3, False)
    c, n, f = _sa_stage(c, n, f, params['sa4'], None, None, None, 3, True)
    feat = f.reshape(-1, 2048)
    return _classifier(feat, params['cls'])


# speedup vs baseline: 1.2541x; 1.2214x over previous
import numpy as np
import jax
import jax.numpy as jnp
from jax.experimental import pallas as pl

_EPS = 1e-8
_F32 = jnp.float32


def _pick_br(rows, mult, cap):
    best = mult
    k = 1
    while True:
        cand = mult * (k + 1)
        if cand > cap or cand > rows:
            break
        k += 1
        if rows % cand == 0:
            best = cand
    return best if rows % best == 0 else rows


def _stats_update(s_ref, y):
    c = y.shape[-1]
    row0 = jnp.sum(y, axis=0, keepdims=True)
    row1 = jnp.sum(y * y, axis=0, keepdims=True)
    upd = jnp.concatenate(
        [row0, row1, jnp.zeros((6, c), dtype=y.dtype)], axis=0)

    @pl.when(pl.program_id(0) == 0)
    def _():
        s_ref[...] = jnp.zeros_like(s_ref)

    s_ref[...] += upd


def _mm0_body(x_ref, wt_ref, b_ref, y_ref, s_ref):
    y = jnp.dot(x_ref[...], wt_ref[...],
                preferred_element_type=_F32) + b_ref[...]
    y_ref[...] = y
    _stats_update(s_ref, y)


def _mm1_body(x_ref, a_ref, c_ref, wt_ref, b_ref, y_ref, s_ref):
    h = jnp.maximum(x_ref[...] * a_ref[...] + c_ref[...], 0.0)
    y = jnp.dot(h, wt_ref[...], preferred_element_type=_F32) + b_ref[...]
    y_ref[...] = y
    _stats_update(s_ref, y)


def _mm2_body(x1_ref, a1_ref, c1_ref, x2_ref, a2_ref, c2_ref,
              wt_ref, b_ref, y_ref, s_ref):
    pre = (x1_ref[...] * a1_ref[...] + c1_ref[...]
           + x2_ref[...] * a2_ref[...] + c2_ref[...])
    h = jnp.maximum(pre, 0.0)
    y = jnp.dot(h, wt_ref[...], preferred_element_type=_F32) + b_ref[...]
    y_ref[...] = y
    _stats_update(s_ref, y)


def _pool_body(x_ref, a_ref, c_ref, o_ref, *, relu, mode):
    h = x_ref[...] * a_ref[...] + c_ref[...]
    if relu:
        h = jnp.maximum(h, 0.0)
    if mode == "max":
        o_ref[...] = jnp.max(h, axis=1)
    else:
        o_ref[...] = jnp.sum(h, axis=1)


def _mm_stats(x, wt, b, affine=None, x2=None, affine2=None):
    rows, cin = x.shape
    cout = wt.shape[1]
    pcin = -(-cin // 128) * 128
    pcout = -(-cout // 128) * 128
    nin = 2 if x2 is not None else 1
    cap = max(8, (4 << 20) // (8 * (nin * pcin + pcout)))
    br = _pick_br(rows, 8, min(cap, rows))
    grid = (rows // br,)
    b2 = b.reshape(1, cout)
    out_shape = [jax.ShapeDtypeStruct((rows, cout), _F32),
                 jax.ShapeDtypeStruct((8, cout), _F32)]
    out_specs = [pl.BlockSpec((br, cout), lambda i: (i, 0)),
                 pl.BlockSpec((8, cout), lambda i: (0, 0))]
    w_spec = pl.BlockSpec((cin, cout), lambda i: (0, 0))
    bias_spec = pl.BlockSpec((1, cout), lambda i: (0, 0))
    row_spec = pl.BlockSpec((br, cin), lambda i: (i, 0))
    vec_spec = pl.BlockSpec((1, cin), lambda i: (0, 0))

    if affine is None:
        y, s = pl.pallas_call(
            _mm0_body, grid=grid,
            in_specs=[row_spec, w_spec, bias_spec],
            out_specs=out_specs, out_shape=out_shape,
        )(x, wt, b2)
    elif x2 is None:
        a, c = affine
        y, s = pl.pallas_call(
            _mm1_body, grid=grid,
            in_specs=[row_spec, vec_spec, vec_spec, w_spec, bias_spec],
            out_specs=out_specs, out_shape=out_shape,
        )(x, a.reshape(1, cin), c.reshape(1, cin), wt, b2)
    else:
        a1, c1 = affine
        a2, c2 = affine2
        y, s = pl.pallas_call(
            _mm2_body, grid=grid,
            in_specs=[row_spec, vec_spec, vec_spec,
                      row_spec, vec_spec, vec_spec, w_spec, bias_spec],
            out_specs=out_specs, out_shape=out_shape,
        )(x, a1.reshape(1, cin), c1.reshape(1, cin),
          x2, a2.reshape(1, cin), c2.reshape(1, cin), wt, b2)
    return y, s


def _bn_affine(stats, rows, gamma, beta):
    mean = stats[0] / rows
    var = stats[1] / rows - mean * mean
    a = gamma / jnp.sqrt(var + 1e-5)
    c = beta - mean * a
    return a, c


def _pool(x, a, c, k, relu, mode):
    rows, cin = x.shape
    groups = rows // k
    x3 = x.reshape(groups, k, cin)
    pcin = -(-cin // 128) * 128
    gb = _pick_br(groups, 8, max(8, (1 << 21) // (4 * k * pcin)))
    if groups % gb != 0 or groups < 8:
        gb = groups
    y = pl.pallas_call(
        lambda x_ref, a_ref, c_ref, o_ref: _pool_body(
            x_ref, a_ref, c_ref, o_ref, relu=relu, mode=mode),
        grid=(groups // gb,),
        in_specs=[pl.BlockSpec((gb, k, cin), lambda i: (i, 0, 0)),
                  pl.BlockSpec((1, 1, cin), lambda i: (0, 0, 0)),
                  pl.BlockSpec((1, 1, cin), lambda i: (0, 0, 0))],
        out_specs=pl.BlockSpec((gb, cin), lambda i: (i, 0)),
        out_shape=jax.ShapeDtypeStruct((groups, cin), _F32),
    )(x3, a.reshape(1, 1, cin), c.reshape(1, 1, cin))
    return y


def _cls_body(x_ref, w1_ref, b1_ref, g1_ref, t1_ref,
              w2_ref, b2_ref, g2_ref, t2_ref,
              w3_ref, b3_ref, o_ref):
    def bn_relu(y, g, t):
        mean = jnp.mean(y, axis=0, keepdims=True)
        var = jnp.mean(y * y, axis=0, keepdims=True) - mean * mean
        return jnp.maximum(g * (y - mean) / jnp.sqrt(var + 1e-5) + t, 0.0)

    h = jnp.dot(x_ref[...], w1_ref[...], preferred_element_type=_F32) \
        + b1_ref[...]
    h = bn_relu(h, g1_ref[...], t1_ref[...])
    h = jnp.dot(h, w2_ref[...], preferred_element_type=_F32) + b2_ref[...]
    h = bn_relu(h, g2_ref[...], t2_ref[...])
    z = jnp.dot(h, w3_ref[...], preferred_element_type=_F32) + b3_ref[...]
    m = jnp.max(z, axis=-1, keepdims=True)
    lse = m + jnp.log(jnp.sum(jnp.exp(z - m), axis=-1, keepdims=True))
    o_ref[...] = z - lse


def _classifier(feat, cp):
    b = feat.shape[0]
    args = (feat,
            cp['W1'].T, cp['b1'].reshape(1, -1),
            cp['g1'].reshape(1, -1), cp['bt1'].reshape(1, -1),
            cp['W2'].T, cp['b2'].reshape(1, -1),
            cp['g2'].reshape(1, -1), cp['bt2'].reshape(1, -1),
            cp['W3'].T, cp['b3'].reshape(1, -1))
    return pl.pallas_call(
        _cls_body,
        out_shape=jax.ShapeDtypeStruct((b, cp['W3'].shape[0]), _F32),
    )(*args)


def _xyz2sphere(xyz):
    rho = jnp.sqrt(jnp.sum(xyz ** 2, axis=-1, keepdims=True))
    rho_c = jnp.maximum(rho, _EPS)
    theta = jnp.arccos(jnp.clip(xyz[..., 2:3] / rho_c, -1.0, 1.0)) / np.pi
    phi = jnp.arctan2(xyz[..., 1:2], xyz[..., 0:1]) / (2 * np.pi) + 0.5
    return jnp.concatenate([rho, theta, phi], axis=-1)


def _index_points(points, idx):
    return jax.vmap(lambda p, i: p[i])(points, idx)


def _sq_dist(src, dst):
    return jnp.sum((src[:, :, None, :] - dst[:, None, :, :]) ** 2, axis=-1)


def _fps_body(xt_ref, o_ref, *, npoint):
    b, _, n = xt_ref.shape
    xt = xt_ref[...]
    iota = jax.lax.broadcasted_iota(jnp.int32, (b, n), 1)
    col = jax.lax.broadcasted_iota(jnp.int32, (b, npoint), 1)

    def step(i, carry):
        dists, idxs = carry
        m = jnp.max(dists, axis=1, keepdims=True)
        far = jnp.min(jnp.where(dists == m, iota, n), axis=1, keepdims=True)
        idxs = jnp.where(col == i, jnp.broadcast_to(far, (b, npoint)), idxs)
        oh = jnp.broadcast_to(far, (b, n)) == iota
        xf = jnp.sum(jnp.where(oh[:, None, :], xt, 0.0), axis=2)
        dx = xt[:, 0, :] - xf[:, 0:1]
        dy = xt[:, 1, :] - xf[:, 1:2]
        dz = xt[:, 2, :] - xf[:, 2:3]
        d = dx * dx + dy * dy + dz * dz
        dists = jnp.minimum(dists, d)
        return dists, idxs

    row = jax.lax.broadcasted_iota(jnp.int32, (b, npoint), 0)
    init = (jnp.full((b, n), 1e10, dtype=_F32),
            jnp.minimum(col + row, 0))
    o_ref[...] = jax.lax.fori_loop(0, npoint, step, init)[1]


def _fps(xyz, npoint):
    b, n, _ = xyz.shape
    xt = jnp.transpose(xyz, (0, 2, 1))
    return pl.pallas_call(
        lambda xt_ref, o_ref: _fps_body(xt_ref, o_ref, npoint=npoint),
        out_shape=jax.ShapeDtypeStruct((b, npoint), jnp.int32),
    )(xt)


def _ball_query(radius, nsample, xyz, new_xyz):
    b, n, _ = xyz.shape
    s = new_xyz.shape[1]
    sqrdists = _sq_dist(new_xyz, xyz)
    gidx = jnp.broadcast_to(jnp.arange(n, dtype=jnp.int32), (b, s, n))
    gidx = jnp.where(sqrdists > radius ** 2, n, gidx)
    gidx = jnp.sort(gidx, axis=-1)[:, :, :nsample]
    first = gidx[:, :, 0:1]
    return jnp.where(gidx == n, first, gidx)


def _knn(k, xyz):
    d = _sq_dist(xyz, xyz)
    return jnp.argsort(d, axis=-1)[:, :, :k]


def _umbrella_surface(center, p, k):
    b, n, _ = center.shape
    idx = _knn(k, center)
    gxyz = _index_points(center, idx)[:, :, 1:]
    gnorm = gxyz - center[:, :, None, :]
    phi = _xyz2sphere(gnorm)[..., 2]
    sidx = jnp.argsort(phi, axis=-1)
    sg = jnp.take_along_axis(
        gnorm, jnp.broadcast_to(sidx[..., None], gnorm.shape), axis=2
    )[:, :, :, None, :]
    sgr = jnp.roll(sg, -1, axis=2)
    g = jnp.concatenate([jnp.zeros_like(sg), sg, sgr], axis=-2)

    e1 = g[..., 1, :] - g[..., 0, :]
    e2 = g[..., 2, :] - g[..., 0, :]
    nor = jnp.cross(e1, e2)
    unit = nor / jnp.maximum(
        jnp.linalg.norm(nor, axis=-1, keepdims=True), _EPS)
    pos_mask = (unit[..., 0:1, 0] > 0).astype(_F32) * 2.0 - 1.0
    unit = unit * pos_mask[..., None]
    gc = jnp.mean(g, axis=-2)
    gpolar = _xyz2sphere(gc)
    gpos = jnp.sum(unit * gc, axis=-1, keepdims=True) / np.sqrt(3.0)
    feat = jnp.concatenate([gc, gpolar, unit, gpos], axis=-1)
    feat = jnp.nan_to_num(feat)

    kk = k - 1
    rows = b * n * kk
    x = feat.reshape(rows, feat.shape[-1])
    y1, s1 = _mm_stats(x, p['W1'].T, p['b1'])
    a1, c1 = _bn_affine(s1, rows, p['g1'], p['bt1'])
    y2, _ = _mm_stats(y1, p['W2'].T, p['b2'], affine=(a1, c1))
    out = _pool(y2, jnp.ones_like(p['b2']), jnp.zeros_like(p['b2']),
                kk, relu=False, mode="sum")
    return out.reshape(b, n, -1)


def _sa_stage(center, normal, feature, p, npoint, radius, nsample,
              pos_channel, group_all):
    b = center.shape[0]
    if group_all:
        new_center = jnp.zeros((b, 1, 3), dtype=center.dtype)
        new_normal = jnp.zeros((b, 1, normal.shape[-1]), dtype=normal.dtype)
        parts = [center[:, None], normal[:, None]]
        if feature is not None:
            parts.append(feature[:, None])
        nf = jnp.concatenate(parts, axis=-1)
        k = center.shape[1]
    else:
        fps_idx = _fps(center, npoint)
        new_center = _index_points(center, fps_idx)
        new_normal = _index_points(normal, fps_idx)
        idx = _ball_query(radius, nsample, center, new_center)
        gpos = _index_points(center, idx) - new_center[:, :, None, :]
        parts = [gpos, _index_points(normal, idx)]
        if feature is not None:
            parts.append(_index_points(feature, idx))
        nf = jnp.concatenate(parts, axis=-1)
        k = nsample

    s = nf.shape[1]
    rows = b * s * k
    cin = nf.shape[-1]
    flat = nf.reshape(rows, cin)
    xp = flat[:, :pos_channel]
    xf = flat[:, pos_channel:]

    yl, sl = _mm_stats(xp, p['Wl0'].T, p['bl0'])
    yf, sf = _mm_stats(xf, p['Wf0'].T, p['bf0'])
    al, cl = _bn_affine(sl, rows, p['gl0'], p['btl0'])
    af, cf = _bn_affine(sf, rows, p['gf0'], p['btf0'])

    y = None
    aff = None
    first = True
    for lay in p['layers']:
        if first:
            y, st = _mm_stats(yl, lay['W'].T, lay['b'],
                              affine=(al, cl), x2=yf, affine2=(af, cf))
            first = False
        else:
            y, st = _mm_stats(y, lay['W'].T, lay['b'], affine=aff)
        aff = _bn_affine(st, rows, lay['g'], lay['bt'])

    f = _pool(y, aff[0], aff[1], k, relu=True, mode="max")
    return new_center, new_normal, f.reshape(b, s, -1)


def kernel(points, params):
    center = jnp.transpose(points[:, :3, :], (0, 2, 1))
    normal = _umbrella_surface(center, params['umb'], 9)
    c, n, f = _sa_stage(center, normal, None, params['sa1'],
                        512, 0.1, 24, 3, False)
    c, n, f = _sa_stage(c, n, f, params['sa2'], 128, 0.2, 24, 3, False)
    c, n, f = _sa_stage(c, n, f, params['sa3'], 32, 0.4, 24, 3, False)
    c, n, f = _sa_stage(c, n, f, params['sa4'], None, None, None, 3, True)
    feat = f.reshape(-1, 2048)
    return _classifier(feat, params['cls'])


# speedup vs baseline: 1.4829x; 1.1824x over previous
import numpy as np
import jax
import jax.numpy as jnp
from jax.experimental import pallas as pl

_EPS = 1e-8
_F32 = jnp.float32


def _pick_br(rows, mult, cap):
    best = mult
    k = 1
    while True:
        cand = mult * (k + 1)
        if cand > cap or cand > rows:
            break
        k += 1
        if rows % cand == 0:
            best = cand
    return best if rows % best == 0 else rows


def _stats_update(s_ref, y):
    c = y.shape[-1]
    row0 = jnp.sum(y, axis=0, keepdims=True)
    row1 = jnp.sum(y * y, axis=0, keepdims=True)
    upd = jnp.concatenate(
        [row0, row1, jnp.zeros((6, c), dtype=y.dtype)], axis=0)

    @pl.when(pl.program_id(0) == 0)
    def _():
        s_ref[...] = jnp.zeros_like(s_ref)

    s_ref[...] += upd


def _mm0_body(x_ref, wt_ref, b_ref, y_ref, s_ref):
    y = jnp.dot(x_ref[...], wt_ref[...],
                preferred_element_type=_F32) + b_ref[...]
    y_ref[...] = y
    _stats_update(s_ref, y)


def _mm1_body(x_ref, a_ref, c_ref, wt_ref, b_ref, y_ref, s_ref):
    h = jnp.maximum(x_ref[...] * a_ref[...] + c_ref[...], 0.0)
    y = jnp.dot(h, wt_ref[...], preferred_element_type=_F32) + b_ref[...]
    y_ref[...] = y
    _stats_update(s_ref, y)


def _mm2_body(x1_ref, a1_ref, c1_ref, x2_ref, a2_ref, c2_ref,
              wt_ref, b_ref, y_ref, s_ref):
    pre = (x1_ref[...] * a1_ref[...] + c1_ref[...]
           + x2_ref[...] * a2_ref[...] + c2_ref[...])
    h = jnp.maximum(pre, 0.0)
    y = jnp.dot(h, wt_ref[...], preferred_element_type=_F32) + b_ref[...]
    y_ref[...] = y
    _stats_update(s_ref, y)


def _pool_body(x_ref, a_ref, c_ref, o_ref, *, relu, mode):
    h = x_ref[...] * a_ref[...] + c_ref[...]
    if relu:
        h = jnp.maximum(h, 0.0)
    if mode == "max":
        o_ref[...] = jnp.max(h, axis=1)
    else:
        o_ref[...] = jnp.sum(h, axis=1)


def _mm_stats(x, wt, b, affine=None, x2=None, affine2=None):
    rows, cin = x.shape
    cout = wt.shape[1]
    pcin = -(-cin // 128) * 128
    pcout = -(-cout // 128) * 128
    nin = 2 if x2 is not None else 1
    cap = max(8, (4 << 20) // (8 * (nin * pcin + pcout)))
    br = _pick_br(rows, 8, min(cap, rows))
    grid = (rows // br,)
    b2 = b.reshape(1, cout)
    out_shape = [jax.ShapeDtypeStruct((rows, cout), _F32),
                 jax.ShapeDtypeStruct((8, cout), _F32)]
    out_specs = [pl.BlockSpec((br, cout), lambda i: (i, 0)),
                 pl.BlockSpec((8, cout), lambda i: (0, 0))]
    w_spec = pl.BlockSpec((cin, cout), lambda i: (0, 0))
    bias_spec = pl.BlockSpec((1, cout), lambda i: (0, 0))
    row_spec = pl.BlockSpec((br, cin), lambda i: (i, 0))
    vec_spec = pl.BlockSpec((1, cin), lambda i: (0, 0))

    if affine is None:
        y, s = pl.pallas_call(
            _mm0_body, grid=grid,
            in_specs=[row_spec, w_spec, bias_spec],
            out_specs=out_specs, out_shape=out_shape,
        )(x, wt, b2)
    elif x2 is None:
        a, c = affine
        y, s = pl.pallas_call(
            _mm1_body, grid=grid,
            in_specs=[row_spec, vec_spec, vec_spec, w_spec, bias_spec],
            out_specs=out_specs, out_shape=out_shape,
        )(x, a.reshape(1, cin), c.reshape(1, cin), wt, b2)
    else:
        a1, c1 = affine
        a2, c2 = affine2
        y, s = pl.pallas_call(
            _mm2_body, grid=grid,
            in_specs=[row_spec, vec_spec, vec_spec,
                      row_spec, vec_spec, vec_spec, w_spec, bias_spec],
            out_specs=out_specs, out_shape=out_shape,
        )(x, a1.reshape(1, cin), c1.reshape(1, cin),
          x2, a2.reshape(1, cin), c2.reshape(1, cin), wt, b2)
    return y, s


def _bn_affine(stats, rows, gamma, beta):
    mean = stats[0] / rows
    var = stats[1] / rows - mean * mean
    a = gamma / jnp.sqrt(var + 1e-5)
    c = beta - mean * a
    return a, c


def _pool(x, a, c, k, relu, mode):
    rows, cin = x.shape
    groups = rows // k
    x3 = x.reshape(groups, k, cin)
    pcin = -(-cin // 128) * 128
    gb = _pick_br(groups, 8, max(8, (1 << 21) // (4 * k * pcin)))
    if groups % gb != 0 or groups < 8:
        gb = groups
    y = pl.pallas_call(
        lambda x_ref, a_ref, c_ref, o_ref: _pool_body(
            x_ref, a_ref, c_ref, o_ref, relu=relu, mode=mode),
        grid=(groups // gb,),
        in_specs=[pl.BlockSpec((gb, k, cin), lambda i: (i, 0, 0)),
                  pl.BlockSpec((1, 1, cin), lambda i: (0, 0, 0)),
                  pl.BlockSpec((1, 1, cin), lambda i: (0, 0, 0))],
        out_specs=pl.BlockSpec((gb, cin), lambda i: (i, 0)),
        out_shape=jax.ShapeDtypeStruct((groups, cin), _F32),
    )(x3, a.reshape(1, 1, cin), c.reshape(1, 1, cin))
    return y


def _cls_body(x_ref, w1_ref, b1_ref, g1_ref, t1_ref,
              w2_ref, b2_ref, g2_ref, t2_ref,
              w3_ref, b3_ref, o_ref):
    def bn_relu(y, g, t):
        mean = jnp.mean(y, axis=0, keepdims=True)
        var = jnp.mean(y * y, axis=0, keepdims=True) - mean * mean
        return jnp.maximum(g * (y - mean) / jnp.sqrt(var + 1e-5) + t, 0.0)

    h = jnp.dot(x_ref[...], w1_ref[...], preferred_element_type=_F32) \
        + b1_ref[...]
    h = bn_relu(h, g1_ref[...], t1_ref[...])
    h = jnp.dot(h, w2_ref[...], preferred_element_type=_F32) + b2_ref[...]
    h = bn_relu(h, g2_ref[...], t2_ref[...])
    z = jnp.dot(h, w3_ref[...], preferred_element_type=_F32) + b3_ref[...]
    m = jnp.max(z, axis=-1, keepdims=True)
    lse = m + jnp.log(jnp.sum(jnp.exp(z - m), axis=-1, keepdims=True))
    o_ref[...] = z - lse


def _classifier(feat, cp):
    b = feat.shape[0]
    args = (feat,
            cp['W1'].T, cp['b1'].reshape(1, -1),
            cp['g1'].reshape(1, -1), cp['bt1'].reshape(1, -1),
            cp['W2'].T, cp['b2'].reshape(1, -1),
            cp['g2'].reshape(1, -1), cp['bt2'].reshape(1, -1),
            cp['W3'].T, cp['b3'].reshape(1, -1))
    return pl.pallas_call(
        _cls_body,
        out_shape=jax.ShapeDtypeStruct((b, cp['W3'].shape[0]), _F32),
    )(*args)


def _xyz2sphere(xyz):
    rho = jnp.sqrt(jnp.sum(xyz ** 2, axis=-1, keepdims=True))
    rho_c = jnp.maximum(rho, _EPS)
    theta = jnp.arccos(jnp.clip(xyz[..., 2:3] / rho_c, -1.0, 1.0)) / np.pi
    phi = jnp.arctan2(xyz[..., 1:2], xyz[..., 0:1]) / (2 * np.pi) + 0.5
    return jnp.concatenate([rho, theta, phi], axis=-1)


def _index_points(points, idx):
    return jax.vmap(lambda p, i: p[i])(points, idx)


def _sq_dist(src, dst):
    return jnp.sum((src[:, :, None, :] - dst[:, None, :, :]) ** 2, axis=-1)


def _fps_body(xt_ref, o_ref, *, npoint):
    b, _, n = xt_ref.shape
    xt = xt_ref[...]
    iota = jax.lax.broadcasted_iota(jnp.int32, (b, n), 1)
    col = jax.lax.broadcasted_iota(jnp.int32, (b, npoint), 1)

    def step(i, carry):
        dists, idxs = carry
        m = jnp.max(dists, axis=1, keepdims=True)
        far = jnp.min(jnp.where(dists == m, iota, n), axis=1, keepdims=True)
        idxs = jnp.where(col == i, jnp.broadcast_to(far, (b, npoint)), idxs)
        oh = jnp.broadcast_to(far, (b, n)) == iota
        xf = jnp.sum(jnp.where(oh[:, None, :], xt, 0.0), axis=2)
        dx = xt[:, 0, :] - xf[:, 0:1]
        dy = xt[:, 1, :] - xf[:, 1:2]
        dz = xt[:, 2, :] - xf[:, 2:3]
        d = dx * dx + dy * dy + dz * dz
        dists = jnp.minimum(dists, d)
        return dists, idxs

    row = jax.lax.broadcasted_iota(jnp.int32, (b, npoint), 0)
    init = (jnp.full((b, n), 1e10, dtype=_F32),
            jnp.minimum(col + row, 0))
    o_ref[...] = jax.lax.fori_loop(0, npoint, step, init)[1]


def _fps(xyz, npoint):
    b, n, _ = xyz.shape
    xt = jnp.transpose(xyz, (0, 2, 1))
    return pl.pallas_call(
        lambda xt_ref, o_ref: _fps_body(xt_ref, o_ref, npoint=npoint),
        out_shape=jax.ShapeDtypeStruct((b, npoint), jnp.int32),
    )(xt)


def _pair_dists(q_ref, xt_ref):
    q = q_ref[0]
    dx = q[:, 0:1] - xt_ref[0, 0:1, :]
    dy = q[:, 1:2] - xt_ref[0, 1:2, :]
    dz = q[:, 2:3] - xt_ref[0, 2:3, :]
    return dx * dx + dy * dy + dz * dz


def _concrete_zero_i32(shape):
    a = jax.lax.broadcasted_iota(jnp.int32, shape, 0)
    b = jax.lax.broadcasted_iota(jnp.int32, shape, 1)
    return jnp.minimum(a + b, 0)


def _knn_body(q_ref, xt_ref, o_ref, *, k):
    qn, n = q_ref.shape[1], xt_ref.shape[2]
    d = _pair_dists(q_ref, xt_ref)
    iota = jax.lax.broadcasted_iota(jnp.int32, (qn, n), 1)
    colk = jax.lax.broadcasted_iota(jnp.int32, (qn, k), 1)
    out = _concrete_zero_i32((qn, k))

    def round_(i, carry):
        d, out = carry
        m = jnp.min(d, axis=1, keepdims=True)
        j = jnp.min(jnp.where(d == m, iota, n), axis=1, keepdims=True)
        out = jnp.where(colk == i, jnp.broadcast_to(j, (qn, k)), out)
        d = jnp.where(jnp.broadcast_to(j, (qn, n)) == iota, jnp.inf, d)
        return d, out

    o_ref[0] = jax.lax.fori_loop(0, k, round_, (d, out))[1]


def _knn(k, xyz):
    b, n, _ = xyz.shape
    xt = jnp.transpose(xyz, (0, 2, 1))
    qb = min(n, 256)
    return pl.pallas_call(
        lambda q_ref, xt_ref, o_ref: _knn_body(q_ref, xt_ref, o_ref, k=k),
        grid=(b, n // qb),
        in_specs=[pl.BlockSpec((1, qb, 3), lambda i, j: (i, j, 0)),
                  pl.BlockSpec((1, 3, n), lambda i, j: (i, 0, 0))],
        out_specs=pl.BlockSpec((1, qb, k), lambda i, j: (i, j, 0)),
        out_shape=jax.ShapeDtypeStruct((b, n, k), jnp.int32),
    )(xyz, xt)


def _ball_body(q_ref, xt_ref, o_ref, *, nsample, r2):
    qn, n = q_ref.shape[1], xt_ref.shape[2]
    d = _pair_dists(q_ref, xt_ref)
    iota = jax.lax.broadcasted_iota(jnp.int32, (qn, n), 1)
    colk = jax.lax.broadcasted_iota(jnp.int32, (qn, nsample), 1)
    key = jnp.where(d > r2, n, iota)
    out = _concrete_zero_i32((qn, nsample))

    def round_(i, carry):
        key, out = carry
        j = jnp.min(key, axis=1, keepdims=True)
        out = jnp.where(colk == i, jnp.broadcast_to(j, (qn, nsample)), out)
        key = jnp.where(jnp.broadcast_to(j, (qn, n)) == iota, n, key)
        return key, out

    out = jax.lax.fori_loop(0, nsample, round_, (key, out))[1]
    first = jnp.broadcast_to(out[:, 0:1], (qn, nsample))
    o_ref[0] = jnp.where(out == n, first, out)


def _ball_query(radius, nsample, xyz, new_xyz):
    b, n, _ = xyz.shape
    s = new_xyz.shape[1]
    xt = jnp.transpose(xyz, (0, 2, 1))
    qb = min(s, 512)
    return pl.pallas_call(
        lambda q_ref, xt_ref, o_ref: _ball_body(
            q_ref, xt_ref, o_ref, nsample=nsample, r2=radius ** 2),
        grid=(b, s // qb),
        in_specs=[pl.BlockSpec((1, qb, 3), lambda i, j: (i, j, 0)),
                  pl.BlockSpec((1, 3, n), lambda i, j: (i, 0, 0))],
        out_specs=pl.BlockSpec((1, qb, nsample), lambda i, j: (i, j, 0)),
        out_shape=jax.ShapeDtypeStruct((b, s, nsample), jnp.int32),
    )(new_xyz, xt)


def _umbrella_surface(center, p, k):
    b, n, _ = center.shape
    idx = _knn(k, center)
    gxyz = _index_points(center, idx)[:, :, 1:]
    gnorm = gxyz - center[:, :, None, :]
    phi = _xyz2sphere(gnorm)[..., 2]
    sidx = jnp.argsort(phi, axis=-1)
    sg = jnp.take_along_axis(
        gnorm, jnp.broadcast_to(sidx[..., None], gnorm.shape), axis=2
    )[:, :, :, None, :]
    sgr = jnp.roll(sg, -1, axis=2)
    g = jnp.concatenate([jnp.zeros_like(sg), sg, sgr], axis=-2)

    e1 = g[..., 1, :] - g[..., 0, :]
    e2 = g[..., 2, :] - g[..., 0, :]
    nor = jnp.cross(e1, e2)
    unit = nor / jnp.maximum(
        jnp.linalg.norm(nor, axis=-1, keepdims=True), _EPS)
    pos_mask = (unit[..., 0:1, 0] > 0).astype(_F32) * 2.0 - 1.0
    unit = unit * pos_mask[..., None]
    gc = jnp.mean(g, axis=-2)
    gpolar = _xyz2sphere(gc)
    gpos = jnp.sum(unit * gc, axis=-1, keepdims=True) / np.sqrt(3.0)
    feat = jnp.concatenate([gc, gpolar, unit, gpos], axis=-1)
    feat = jnp.nan_to_num(feat)

    kk = k - 1
    rows = b * n * kk
    x = feat.reshape(rows, feat.shape[-1])
    y1, s1 = _mm_stats(x, p['W1'].T, p['b1'])
    a1, c1 = _bn_affine(s1, rows, p['g1'], p['bt1'])
    y2, _ = _mm_stats(y1, p['W2'].T, p['b2'], affine=(a1, c1))
    out = _pool(y2, jnp.ones_like(p['b2']), jnp.zeros_like(p['b2']),
                kk, relu=False, mode="sum")
    return out.reshape(b, n, -1)


def _sa_stage(center, normal, feature, p, npoint, radius, nsample,
              pos_channel, group_all):
    b = center.shape[0]
    if group_all:
        new_center = jnp.zeros((b, 1, 3), dtype=center.dtype)
        new_normal = jnp.zeros((b, 1, normal.shape[-1]), dtype=normal.dtype)
        parts = [center[:, None], normal[:, None]]
        if feature is not None:
            parts.append(feature[:, None])
        nf = jnp.concatenate(parts, axis=-1)
        k = center.shape[1]
    else:
        fps_idx = _fps(center, npoint)
        new_center = _index_points(center, fps_idx)
        new_normal = _index_points(normal, fps_idx)
        idx = _ball_query(radius, nsample, center, new_center)
        gpos = _index_points(center, idx) - new_center[:, :, None, :]
        parts = [gpos, _index_points(normal, idx)]
        if feature is not None:
            parts.append(_index_points(feature, idx))
        nf = jnp.concatenate(parts, axis=-1)
        k = nsample

    s = nf.shape[1]
    rows = b * s * k
    cin = nf.shape[-1]
    flat = nf.reshape(rows, cin)
    xp = flat[:, :pos_channel]
    xf = flat[:, pos_channel:]

    yl, sl = _mm_stats(xp, p['Wl0'].T, p['bl0'])
    yf, sf = _mm_stats(xf, p['Wf0'].T, p['bf0'])
    al, cl = _bn_affine(sl, rows, p['gl0'], p['btl0'])
    af, cf = _bn_affine(sf, rows, p['gf0'], p['btf0'])

    y = None
    aff = None
    first = True
    for lay in p['layers']:
        if first:
            y, st = _mm_stats(yl, lay['W'].T, lay['b'],
                              affine=(al, cl), x2=yf, affine2=(af, cf))
            first = False
        else:
            y, st = _mm_stats(y, lay['W'].T, lay['b'], affine=aff)
        aff = _bn_affine(st, rows, lay['g'], lay['bt'])

    f = _pool(y, aff[0], aff[1], k, relu=True, mode="max")
    return new_center, new_normal, f.reshape(b, s, -1)


def kernel(points, params):
    center = jnp.transpose(points[:, :3, :], (0, 2, 1))
    normal = _umbrella_surface(center, params['umb'], 9)
    c, n, f = _sa_stage(center, normal, None, params['sa1'],
                        512, 0.1, 24, 3, False)
    c, n, f = _sa_stage(c, n, f, params['sa2'], 128, 0.2, 24, 3, False)
    c, n, f = _sa_stage(c, n, f, params['sa3'], 32, 0.4, 24, 3, False)
    c, n, f = _sa_stage(c, n, f, params['sa4'], None, None, None, 3, True)
    feat = f.reshape(-1, 2048)
    return _classifier(feat, params['cls'])


# speedup vs baseline: 5.8046x; 3.9144x over previous
import functools

import numpy as np
import jax
import jax.numpy as jnp
from jax.experimental import pallas as pl
from jax.experimental.pallas import tpu as pltpu
from jax.experimental.pallas import tpu_sc as plsc

_EPS = 1e-8
_F32 = jnp.float32


def _pick_br(rows, mult, cap):
    best = mult
    k = 1
    while True:
        cand = mult * (k + 1)
        if cand > cap or cand > rows:
            break
        k += 1
        if rows % cand == 0:
            best = cand
    return best if rows % best == 0 else rows


def _stats_update(s_ref, y):
    c = y.shape[-1]
    row0 = jnp.sum(y, axis=0, keepdims=True)
    row1 = jnp.sum(y * y, axis=0, keepdims=True)
    upd = jnp.concatenate(
        [row0, row1, jnp.zeros((6, c), dtype=y.dtype)], axis=0)

    @pl.when(pl.program_id(0) == 0)
    def _():
        s_ref[...] = jnp.zeros_like(s_ref)

    s_ref[...] += upd


def _mm0_body(x_ref, wt_ref, b_ref, y_ref, s_ref):
    y = jnp.dot(x_ref[...], wt_ref[...],
                preferred_element_type=_F32) + b_ref[...]
    y_ref[...] = y
    _stats_update(s_ref, y)


def _mm1_body(x_ref, a_ref, c_ref, wt_ref, b_ref, y_ref, s_ref):
    h = jnp.maximum(x_ref[...] * a_ref[...] + c_ref[...], 0.0)
    y = jnp.dot(h, wt_ref[...], preferred_element_type=_F32) + b_ref[...]
    y_ref[...] = y
    _stats_update(s_ref, y)


def _mm2_body(x1_ref, a1_ref, c1_ref, x2_ref, a2_ref, c2_ref,
              wt_ref, b_ref, y_ref, s_ref):
    pre = (x1_ref[...] * a1_ref[...] + c1_ref[...]
           + x2_ref[...] * a2_ref[...] + c2_ref[...])
    h = jnp.maximum(pre, 0.0)
    y = jnp.dot(h, wt_ref[...], preferred_element_type=_F32) + b_ref[...]
    y_ref[...] = y
    _stats_update(s_ref, y)


def _pool_body(x_ref, a_ref, c_ref, o_ref, *, relu, mode):
    h = x_ref[...] * a_ref[...] + c_ref[...]
    if relu:
        h = jnp.maximum(h, 0.0)
    if mode == "max":
        o_ref[...] = jnp.max(h, axis=1)
    else:
        o_ref[...] = jnp.sum(h, axis=1)


def _mm_stats(x, wt, b, affine=None, x2=None, affine2=None):
    rows, cin = x.shape
    cout = wt.shape[1]
    pcin = -(-cin // 128) * 128
    pcout = -(-cout // 128) * 128
    nin = 2 if x2 is not None else 1
    cap = max(8, (4 << 20) // (8 * (nin * pcin + pcout)))
    br = _pick_br(rows, 8, min(cap, rows))
    grid = (rows // br,)
    b2 = b.reshape(1, cout)
    out_shape = [jax.ShapeDtypeStruct((rows, cout), _F32),
                 jax.ShapeDtypeStruct((8, cout), _F32)]
    out_specs = [pl.BlockSpec((br, cout), lambda i: (i, 0)),
                 pl.BlockSpec((8, cout), lambda i: (0, 0))]
    w_spec = pl.BlockSpec((cin, cout), lambda i: (0, 0))
    bias_spec = pl.BlockSpec((1, cout), lambda i: (0, 0))
    row_spec = pl.BlockSpec((br, cin), lambda i: (i, 0))
    vec_spec = pl.BlockSpec((1, cin), lambda i: (0, 0))

    if affine is None:
        y, s = pl.pallas_call(
            _mm0_body, grid=grid,
            in_specs=[row_spec, w_spec, bias_spec],
            out_specs=out_specs, out_shape=out_shape,
        )(x, wt, b2)
    elif x2 is None:
        a, c = affine
        y, s = pl.pallas_call(
            _mm1_body, grid=grid,
            in_specs=[row_spec, vec_spec, vec_spec, w_spec, bias_spec],
            out_specs=out_specs, out_shape=out_shape,
        )(x, a.reshape(1, cin), c.reshape(1, cin), wt, b2)
    else:
        a1, c1 = affine
        a2, c2 = affine2
        y, s = pl.pallas_call(
            _mm2_body, grid=grid,
            in_specs=[row_spec, vec_spec, vec_spec,
                      row_spec, vec_spec, vec_spec, w_spec, bias_spec],
            out_specs=out_specs, out_shape=out_shape,
        )(x, a1.reshape(1, cin), c1.reshape(1, cin),
          x2, a2.reshape(1, cin), c2.reshape(1, cin), wt, b2)
    return y, s


def _bn_affine(stats, rows, gamma, beta):
    mean = stats[0] / rows
    var = stats[1] / rows - mean * mean
    a = gamma / jnp.sqrt(var + 1e-5)
    c = beta - mean * a
    return a, c


def _pool(x, a, c, k, relu, mode):
    rows, cin = x.shape
    groups = rows // k
    x3 = x.reshape(groups, k, cin)
    pcin = -(-cin // 128) * 128
    gb = _pick_br(groups, 8, max(8, (1 << 21) // (4 * k * pcin)))
    if groups % gb != 0 or groups < 8:
        gb = groups
    y = pl.pallas_call(
        lambda x_ref, a_ref, c_ref, o_ref: _pool_body(
            x_ref, a_ref, c_ref, o_ref, relu=relu, mode=mode),
        grid=(groups // gb,),
        in_specs=[pl.BlockSpec((gb, k, cin), lambda i: (i, 0, 0)),
                  pl.BlockSpec((1, 1, cin), lambda i: (0, 0, 0)),
                  pl.BlockSpec((1, 1, cin), lambda i: (0, 0, 0))],
        out_specs=pl.BlockSpec((gb, cin), lambda i: (i, 0)),
        out_shape=jax.ShapeDtypeStruct((groups, cin), _F32),
    )(x3, a.reshape(1, 1, cin), c.reshape(1, 1, cin))
    return y


def _cls_body(x_ref, w1_ref, b1_ref, g1_ref, t1_ref,
              w2_ref, b2_ref, g2_ref, t2_ref,
              w3_ref, b3_ref, o_ref):
    def bn_relu(y, g, t):
        mean = jnp.mean(y, axis=0, keepdims=True)
        var = jnp.mean(y * y, axis=0, keepdims=True) - mean * mean
        return jnp.maximum(g * (y - mean) / jnp.sqrt(var + 1e-5) + t, 0.0)

    h = jnp.dot(x_ref[...], w1_ref[...], preferred_element_type=_F32) \
        + b1_ref[...]
    h = bn_relu(h, g1_ref[...], t1_ref[...])
    h = jnp.dot(h, w2_ref[...], preferred_element_type=_F32) + b2_ref[...]
    h = bn_relu(h, g2_ref[...], t2_ref[...])
    z = jnp.dot(h, w3_ref[...], preferred_element_type=_F32) + b3_ref[...]
    m = jnp.max(z, axis=-1, keepdims=True)
    lse = m + jnp.log(jnp.sum(jnp.exp(z - m), axis=-1, keepdims=True))
    o_ref[...] = z - lse


def _classifier(feat, cp):
    b = feat.shape[0]
    args = (feat,
            cp['W1'].T, cp['b1'].reshape(1, -1),
            cp['g1'].reshape(1, -1), cp['bt1'].reshape(1, -1),
            cp['W2'].T, cp['b2'].reshape(1, -1),
            cp['g2'].reshape(1, -1), cp['bt2'].reshape(1, -1),
            cp['W3'].T, cp['b3'].reshape(1, -1))
    return pl.pallas_call(
        _cls_body,
        out_shape=jax.ShapeDtypeStruct((b, cp['W3'].shape[0]), _F32),
    )(*args)


_SC_CH = 128
_SC_NC = 2
_SC_NS = 16
_SC_NW = _SC_NC * _SC_NS


def _sc_gather_call(rows, d, n_chunks):
    mesh = plsc.VectorSubcoreMesh(core_axis_name="c", subcore_axis_name="s")
    nc = _SC_NC
    b_per_w = n_chunks * _SC_CH

    @functools.partial(
        pl.kernel, mesh=mesh,
        out_type=jax.ShapeDtypeStruct((rows, d), jnp.float32),
        scratch_types=[pltpu.VMEM((_SC_CH,), jnp.int32),
                       pltpu.VMEM((_SC_CH, d), jnp.float32),
                       pltpu.SemaphoreType.DMA],
    )
    def gather_k(table_hbm, idx_hbm, out_hbm, idx_v, rows_v, sem):
        wid = jax.lax.axis_index("s") * nc + jax.lax.axis_index("c")
        base = wid * b_per_w

        def body(i, carry):
            off = base + i * _SC_CH
            pltpu.sync_copy(idx_hbm.at[pl.ds(off, _SC_CH)], idx_v)
            pltpu.async_copy(table_hbm.at[idx_v], rows_v, sem).wait()
            pltpu.sync_copy(rows_v, out_hbm.at[pl.ds(off, _SC_CH)])
            return carry

        jax.lax.fori_loop(0, n_chunks, body, 0)

    return gather_k


def _sc_gather(table, idx):
    t, c = table.shape
    d = -(-c // 128) * 128
    if d != c:
        table = jnp.pad(table, ((0, 0), (0, d - c)))
    unit = _SC_NW * _SC_CH
    r = idx.shape[0]
    rp = -(-r // unit) * unit
    if rp != r:
        idx = jnp.pad(idx, (0, rp - r))
    out = _sc_gather_call(rp, d, rp // unit)(table, idx)
    return out[:r, :c]


def _gather_rows(src, idx):
    b, n, c = src.shape
    off = (jnp.arange(b, dtype=jnp.int32) * n).reshape(
        (b,) + (1,) * (idx.ndim - 1))
    flat = _sc_gather(src.reshape(b * n, c),
                      (idx.astype(jnp.int32) + off).reshape(-1))
    return flat.reshape(idx.shape + (c,))


def _xyz2sphere(xyz):
    rho = jnp.sqrt(jnp.sum(xyz ** 2, axis=-1, keepdims=True))
    rho_c = jnp.maximum(rho, _EPS)
    theta = jnp.arccos(jnp.clip(xyz[..., 2:3] / rho_c, -1.0, 1.0)) / np.pi
    phi = jnp.arctan2(xyz[..., 1:2], xyz[..., 0:1]) / (2 * np.pi) + 0.5
    return jnp.concatenate([rho, theta, phi], axis=-1)


def _index_points(points, idx):
    return jax.vmap(lambda p, i: p[i])(points, idx)


def _sq_dist(src, dst):
    return jnp.sum((src[:, :, None, :] - dst[:, None, :, :]) ** 2, axis=-1)


def _fps_body(xt_ref, o_ref, *, npoint):
    b, _, n = xt_ref.shape
    xt = xt_ref[...]
    iota = jax.lax.broadcasted_iota(jnp.int32, (b, n), 1)
    col = jax.lax.broadcasted_iota(jnp.int32, (b, npoint), 1)

    def step(i, carry):
        dists, idxs = carry
        m = jnp.max(dists, axis=1, keepdims=True)
        far = jnp.min(jnp.where(dists == m, iota, n), axis=1, keepdims=True)
        idxs = jnp.where(col == i, jnp.broadcast_to(far, (b, npoint)), idxs)
        oh = jnp.broadcast_to(far, (b, n)) == iota
        xf = jnp.sum(jnp.where(oh[:, None, :], xt, 0.0), axis=2)
        dx = xt[:, 0, :] - xf[:, 0:1]
        dy = xt[:, 1, :] - xf[:, 1:2]
        dz = xt[:, 2, :] - xf[:, 2:3]
        d = dx * dx + dy * dy + dz * dz
        dists = jnp.minimum(dists, d)
        return dists, idxs

    row = jax.lax.broadcasted_iota(jnp.int32, (b, npoint), 0)
    init = (jnp.full((b, n), 1e10, dtype=_F32),
            jnp.minimum(col + row, 0))
    o_ref[...] = jax.lax.fori_loop(0, npoint, step, init)[1]


def _fps(xyz, npoint):
    b, n, _ = xyz.shape
    xt = jnp.transpose(xyz, (0, 2, 1))
    return pl.pallas_call(
        lambda xt_ref, o_ref: _fps_body(xt_ref, o_ref, npoint=npoint),
        out_shape=jax.ShapeDtypeStruct((b, npoint), jnp.int32),
    )(xt)


def _pair_dists(q_ref, xt_ref):
    q = q_ref[0]
    dx = q[:, 0:1] - xt_ref[0, 0:1, :]
    dy = q[:, 1:2] - xt_ref[0, 1:2, :]
    dz = q[:, 2:3] - xt_ref[0, 2:3, :]
    return dx * dx + dy * dy + dz * dz


def _concrete_zero_i32(shape):
    a = jax.lax.broadcasted_iota(jnp.int32, shape, 0)
    b = jax.lax.broadcasted_iota(jnp.int32, shape, 1)
    return jnp.minimum(a + b, 0)


def _knn_body(q_ref, xt_ref, o_ref, *, k):
    qn, n = q_ref.shape[1], xt_ref.shape[2]
    d = _pair_dists(q_ref, xt_ref)
    iota = jax.lax.broadcasted_iota(jnp.int32, (qn, n), 1)
    colk = jax.lax.broadcasted_iota(jnp.int32, (qn, k), 1)
    out = _concrete_zero_i32((qn, k))

    def round_(i, carry):
        d, out = carry
        m = jnp.min(d, axis=1, keepdims=True)
        j = jnp.min(jnp.where(d == m, iota, n), axis=1, keepdims=True)
        out = jnp.where(colk == i, jnp.broadcast_to(j, (qn, k)), out)
        d = jnp.where(jnp.broadcast_to(j, (qn, n)) == iota, jnp.inf, d)
        return d, out

    o_ref[0] = jax.lax.fori_loop(0, k, round_, (d, out))[1]


def _knn(k, xyz):
    b, n, _ = xyz.shape
    xt = jnp.transpose(xyz, (0, 2, 1))
    qb = min(n, 256)
    return pl.pallas_call(
        lambda q_ref, xt_ref, o_ref: _knn_body(q_ref, xt_ref, o_ref, k=k),
        grid=(b, n // qb),
        in_specs=[pl.BlockSpec((1, qb, 3), lambda i, j: (i, j, 0)),
                  pl.BlockSpec((1, 3, n), lambda i, j: (i, 0, 0))],
        out_specs=pl.BlockSpec((1, qb, k), lambda i, j: (i, j, 0)),
        out_shape=jax.ShapeDtypeStruct((b, n, k), jnp.int32),
    )(xyz, xt)


def _ball_body(q_ref, xt_ref, o_ref, *, nsample, r2):
    qn, n = q_ref.shape[1], xt_ref.shape[2]
    d = _pair_dists(q_ref, xt_ref)
    iota = jax.lax.broadcasted_iota(jnp.int32, (qn, n), 1)
    colk = jax.lax.broadcasted_iota(jnp.int32, (qn, nsample), 1)
    key = jnp.where(d > r2, n, iota)
    out = _concrete_zero_i32((qn, nsample))

    def round_(i, carry):
        key, out = carry
        j = jnp.min(key, axis=1, keepdims=True)
        out = jnp.where(colk == i, jnp.broadcast_to(j, (qn, nsample)), out)
        key = jnp.where(jnp.broadcast_to(j, (qn, n)) == iota, n, key)
        return key, out

    out = jax.lax.fori_loop(0, nsample, round_, (key, out))[1]
    first = jnp.broadcast_to(out[:, 0:1], (qn, nsample))
    o_ref[0] = jnp.where(out == n, first, out)


def _ball_query(radius, nsample, xyz, new_xyz):
    b, n, _ = xyz.shape
    s = new_xyz.shape[1]
    xt = jnp.transpose(xyz, (0, 2, 1))
    qb = min(s, 512)
    return pl.pallas_call(
        lambda q_ref, xt_ref, o_ref: _ball_body(
            q_ref, xt_ref, o_ref, nsample=nsample, r2=radius ** 2),
        grid=(b, s // qb),
        in_specs=[pl.BlockSpec((1, qb, 3), lambda i, j: (i, j, 0)),
                  pl.BlockSpec((1, 3, n), lambda i, j: (i, 0, 0))],
        out_specs=pl.BlockSpec((1, qb, nsample), lambda i, j: (i, j, 0)),
        out_shape=jax.ShapeDtypeStruct((b, s, nsample), jnp.int32),
    )(new_xyz, xt)


def _umbrella_surface(center, p, k):
    b, n, _ = center.shape
    idx = _knn(k, center)
    gxyz = _gather_rows(center, idx[:, :, 1:])
    gnorm = gxyz - center[:, :, None, :]
    phi = _xyz2sphere(gnorm)[..., 2]
    sidx = jnp.argsort(phi, axis=-1).astype(jnp.int32)
    kk1 = k - 1
    srows = jnp.arange(n, dtype=jnp.int32)[None, :, None] * kk1 + sidx
    sg = _gather_rows(gnorm.reshape(b, n * kk1, 3), srows)[:, :, :, None, :]
    sgr = jnp.roll(sg, -1, axis=2)
    g = jnp.concatenate([jnp.zeros_like(sg), sg, sgr], axis=-2)

    e1 = g[..., 1, :] - g[..., 0, :]
    e2 = g[..., 2, :] - g[..., 0, :]
    nor = jnp.cross(e1, e2)
    unit = nor / jnp.maximum(
        jnp.linalg.norm(nor, axis=-1, keepdims=True), _EPS)
    pos_mask = (unit[..., 0:1, 0] > 0).astype(_F32) * 2.0 - 1.0
    unit = unit * pos_mask[..., None]
    gc = jnp.mean(g, axis=-2)
    gpolar = _xyz2sphere(gc)
    gpos = jnp.sum(unit * gc, axis=-1, keepdims=True) / np.sqrt(3.0)
    feat = jnp.concatenate([gc, gpolar, unit, gpos], axis=-1)
    feat = jnp.nan_to_num(feat)

    kk = k - 1
    rows = b * n * kk
    x = feat.reshape(rows, feat.shape[-1])
    y1, s1 = _mm_stats(x, p['W1'].T, p['b1'])
    a1, c1 = _bn_affine(s1, rows, p['g1'], p['bt1'])
    y2, _ = _mm_stats(y1, p['W2'].T, p['b2'], affine=(a1, c1))
    out = _pool(y2, jnp.ones_like(p['b2']), jnp.zeros_like(p['b2']),
                kk, relu=False, mode="sum")
    return out.reshape(b, n, -1)


def _sa_stage(center, normal, feature, p, npoint, radius, nsample,
              pos_channel, group_all):
    b = center.shape[0]
    if group_all:
        new_center = jnp.zeros((b, 1, 3), dtype=center.dtype)
        new_normal = jnp.zeros((b, 1, normal.shape[-1]), dtype=normal.dtype)
        parts = [center[:, None], normal[:, None]]
        if feature is not None:
            parts.append(feature[:, None])
        nf = jnp.concatenate(parts, axis=-1)
        k = center.shape[1]
        s = nf.shape[1]
        rows = b * s * k
        cin = nf.shape[-1]
        flat = nf.reshape(rows, cin)
        xp = flat[:, :pos_channel]
        xf = flat[:, pos_channel:]
    else:
        parts = [center, normal]
        if feature is not None:
            parts.append(feature)
        table = jnp.concatenate(parts, axis=-1)
        fps_idx = _fps(center, npoint)
        sampled = _gather_rows(table, fps_idx)
        new_center = sampled[..., :3]
        new_normal = sampled[..., 3:3 + normal.shape[-1]]
        idx = _ball_query(radius, nsample, center, new_center)
        nbr = _gather_rows(table, idx)
        k = nsample
        s = npoint
        rows = b * s * k
        cin = table.shape[-1]
        flat = nbr.reshape(rows, cin)
        xp = (flat[:, :pos_channel]
              - jnp.broadcast_to(new_center[:, :, None, :],
                                 (b, s, k, 3)).reshape(rows, 3))
        xf = flat[:, pos_channel:]

    yl, sl = _mm_stats(xp, p['Wl0'].T, p['bl0'])
    yf, sf = _mm_stats(xf, p['Wf0'].T, p['bf0'])
    al, cl = _bn_affine(sl, rows, p['gl0'], p['btl0'])
    af, cf = _bn_affine(sf, rows, p['gf0'], p['btf0'])

    y = None
    aff = None
    first = True
    for lay in p['layers']:
        if first:
            y, st = _mm_stats(yl, lay['W'].T, lay['b'],
                              affine=(al, cl), x2=yf, affine2=(af, cf))
            first = False
        else:
            y, st = _mm_stats(y, lay['W'].T, lay['b'], affine=aff)
        aff = _bn_affine(st, rows, lay['g'], lay['bt'])

    f = _pool(y, aff[0], aff[1], k, relu=True, mode="max")
    return new_center, new_normal, f.reshape(b, s, -1)


def kernel(points, params):
    center = jnp.transpose(points[:, :3, :], (0, 2, 1))
    normal = _umbrella_surface(center, params['umb'], 9)
    c, n, f = _sa_stage(center, normal, None, params['sa1'],
                        512, 0.1, 24, 3, False)
    c, n, f = _sa_stage(c, n, f, params['sa2'], 128, 0.2, 24, 3, False)
    c, n, f = _sa_stage(c, n, f, params['sa3'], 32, 0.4, 24, 3, False)
    c, n, f = _sa_stage(c, n, f, params['sa4'], None, None, None, 3, True)
    feat = f.reshape(-1, 2048)
    return _classifier(feat, params['cls'])


# speedup vs baseline: 6.7469x; 1.1623x over previous
import functools

import numpy as np
import jax
import jax.numpy as jnp
from jax.experimental import pallas as pl
from jax.experimental.pallas import tpu as pltpu
from jax.experimental.pallas import tpu_sc as plsc

_EPS = 1e-8
_F32 = jnp.float32


def _pick_br(rows, mult, cap):
    best = mult
    k = 1
    while True:
        cand = mult * (k + 1)
        if cand > cap or cand > rows:
            break
        k += 1
        if rows % cand == 0:
            best = cand
    return best if rows % best == 0 else rows


def _stats_update(s_ref, y):
    c = y.shape[-1]
    row0 = jnp.sum(y, axis=0, keepdims=True)
    row1 = jnp.sum(y * y, axis=0, keepdims=True)
    upd = jnp.concatenate(
        [row0, row1, jnp.zeros((6, c), dtype=y.dtype)], axis=0)

    @pl.when(pl.program_id(0) == 0)
    def _():
        s_ref[...] = jnp.zeros_like(s_ref)

    s_ref[...] += upd


def _mm0_body(x_ref, wt_ref, b_ref, y_ref, s_ref):
    y = jnp.dot(x_ref[...], wt_ref[...],
                preferred_element_type=_F32) + b_ref[...]
    y_ref[...] = y
    _stats_update(s_ref, y)


def _mm1_body(x_ref, a_ref, c_ref, wt_ref, b_ref, y_ref, s_ref):
    h = jnp.maximum(x_ref[...] * a_ref[...] + c_ref[...], 0.0)
    y = jnp.dot(h, wt_ref[...], preferred_element_type=_F32) + b_ref[...]
    y_ref[...] = y
    _stats_update(s_ref, y)


def _mm2_body(x1_ref, a1_ref, c1_ref, x2_ref, a2_ref, c2_ref,
              wt_ref, b_ref, y_ref, s_ref):
    pre = (x1_ref[...] * a1_ref[...] + c1_ref[...]
           + x2_ref[...] * a2_ref[...] + c2_ref[...])
    h = jnp.maximum(pre, 0.0)
    y = jnp.dot(h, wt_ref[...], preferred_element_type=_F32) + b_ref[...]
    y_ref[...] = y
    _stats_update(s_ref, y)


def _pool_body(x_ref, a_ref, c_ref, o_ref, *, relu, mode):
    h = x_ref[...] * a_ref[...] + c_ref[...]
    if relu:
        h = jnp.maximum(h, 0.0)
    if mode == "max":
        o_ref[...] = jnp.max(h, axis=1)
    else:
        o_ref[...] = jnp.sum(h, axis=1)


def _mm_stats(x, wt, b, affine=None, x2=None, affine2=None):
    rows, cin = x.shape
    cout = wt.shape[1]
    pcin = -(-cin // 128) * 128
    pcout = -(-cout // 128) * 128
    nin = 2 if x2 is not None else 1
    cap = max(8, (4 << 20) // (8 * (nin * pcin + pcout)))
    br = _pick_br(rows, 8, min(cap, rows))
    grid = (rows // br,)
    b2 = b.reshape(1, cout)
    out_shape = [jax.ShapeDtypeStruct((rows, cout), _F32),
                 jax.ShapeDtypeStruct((8, cout), _F32)]
    out_specs = [pl.BlockSpec((br, cout), lambda i: (i, 0)),
                 pl.BlockSpec((8, cout), lambda i: (0, 0))]
    w_spec = pl.BlockSpec((cin, cout), lambda i: (0, 0))
    bias_spec = pl.BlockSpec((1, cout), lambda i: (0, 0))
    row_spec = pl.BlockSpec((br, cin), lambda i: (i, 0))
    vec_spec = pl.BlockSpec((1, cin), lambda i: (0, 0))

    if affine is None:
        y, s = pl.pallas_call(
            _mm0_body, grid=grid,
            in_specs=[row_spec, w_spec, bias_spec],
            out_specs=out_specs, out_shape=out_shape,
        )(x, wt, b2)
    elif x2 is None:
        a, c = affine
        y, s = pl.pallas_call(
            _mm1_body, grid=grid,
            in_specs=[row_spec, vec_spec, vec_spec, w_spec, bias_spec],
            out_specs=out_specs, out_shape=out_shape,
        )(x, a.reshape(1, cin), c.reshape(1, cin), wt, b2)
    else:
        a1, c1 = affine
        a2, c2 = affine2
        y, s = pl.pallas_call(
            _mm2_body, grid=grid,
            in_specs=[row_spec, vec_spec, vec_spec,
                      row_spec, vec_spec, vec_spec, w_spec, bias_spec],
            out_specs=out_specs, out_shape=out_shape,
        )(x, a1.reshape(1, cin), c1.reshape(1, cin),
          x2, a2.reshape(1, cin), c2.reshape(1, cin), wt, b2)
    return y, s


def _bn_affine(stats, rows, gamma, beta):
    mean = stats[0] / rows
    var = stats[1] / rows - mean * mean
    a = gamma / jnp.sqrt(var + 1e-5)
    c = beta - mean * a
    return a, c


def _pool(x, a, c, k, relu, mode):
    rows, cin = x.shape
    groups = rows // k
    x3 = x.reshape(groups, k, cin)
    pcin = -(-cin // 128) * 128
    gb = _pick_br(groups, 8, max(8, (1 << 21) // (4 * k * pcin)))
    if groups % gb != 0 or groups < 8:
        gb = groups
    y = pl.pallas_call(
        lambda x_ref, a_ref, c_ref, o_ref: _pool_body(
            x_ref, a_ref, c_ref, o_ref, relu=relu, mode=mode),
        grid=(groups // gb,),
        in_specs=[pl.BlockSpec((gb, k, cin), lambda i: (i, 0, 0)),
                  pl.BlockSpec((1, 1, cin), lambda i: (0, 0, 0)),
                  pl.BlockSpec((1, 1, cin), lambda i: (0, 0, 0))],
        out_specs=pl.BlockSpec((gb, cin), lambda i: (i, 0)),
        out_shape=jax.ShapeDtypeStruct((groups, cin), _F32),
    )(x3, a.reshape(1, 1, cin), c.reshape(1, 1, cin))
    return y


def _cls_body(x_ref, w1_ref, b1_ref, g1_ref, t1_ref,
              w2_ref, b2_ref, g2_ref, t2_ref,
              w3_ref, b3_ref, o_ref):
    def bn_relu(y, g, t):
        mean = jnp.mean(y, axis=0, keepdims=True)
        var = jnp.mean(y * y, axis=0, keepdims=True) - mean * mean
        return jnp.maximum(g * (y - mean) / jnp.sqrt(var + 1e-5) + t, 0.0)

    h = jnp.dot(x_ref[...], w1_ref[...], preferred_element_type=_F32) \
        + b1_ref[...]
    h = bn_relu(h, g1_ref[...], t1_ref[...])
    h = jnp.dot(h, w2_ref[...], preferred_element_type=_F32) + b2_ref[...]
    h = bn_relu(h, g2_ref[...], t2_ref[...])
    z = jnp.dot(h, w3_ref[...], preferred_element_type=_F32) + b3_ref[...]
    m = jnp.max(z, axis=-1, keepdims=True)
    lse = m + jnp.log(jnp.sum(jnp.exp(z - m), axis=-1, keepdims=True))
    o_ref[...] = z - lse


def _classifier(feat, cp):
    b = feat.shape[0]
    args = (feat,
            cp['W1'].T, cp['b1'].reshape(1, -1),
            cp['g1'].reshape(1, -1), cp['bt1'].reshape(1, -1),
            cp['W2'].T, cp['b2'].reshape(1, -1),
            cp['g2'].reshape(1, -1), cp['bt2'].reshape(1, -1),
            cp['W3'].T, cp['b3'].reshape(1, -1))
    return pl.pallas_call(
        _cls_body,
        out_shape=jax.ShapeDtypeStruct((b, cp['W3'].shape[0]), _F32),
    )(*args)


_SC_CH = 128
_SC_NC = 2
_SC_NS = 16
_SC_NW = _SC_NC * _SC_NS


def _sc_gather_call(rows, d, n_chunks):
    mesh = plsc.VectorSubcoreMesh(core_axis_name="c", subcore_axis_name="s")
    nc = _SC_NC
    b_per_w = n_chunks * _SC_CH

    @functools.partial(
        pl.kernel, mesh=mesh,
        out_type=jax.ShapeDtypeStruct((rows, d), jnp.float32),
        scratch_types=[pltpu.VMEM((_SC_CH,), jnp.int32),
                       pltpu.VMEM((_SC_CH, d), jnp.float32),
                       pltpu.SemaphoreType.DMA],
    )
    def gather_k(table_hbm, idx_hbm, out_hbm, idx_v, rows_v, sem):
        wid = jax.lax.axis_index("s") * nc + jax.lax.axis_index("c")
        base = wid * b_per_w

        def body(i, carry):
            off = base + i * _SC_CH
            pltpu.sync_copy(idx_hbm.at[pl.ds(off, _SC_CH)], idx_v)
            pltpu.async_copy(table_hbm.at[idx_v], rows_v, sem).wait()
            pltpu.sync_copy(rows_v, out_hbm.at[pl.ds(off, _SC_CH)])
            return carry

        jax.lax.fori_loop(0, n_chunks, body, 0)

    return gather_k


def _sc_gather(table, idx):
    t, c = table.shape
    d = -(-c // 128) * 128
    if d != c:
        table = jnp.pad(table, ((0, 0), (0, d - c)))
    unit = _SC_NW * _SC_CH
    r = idx.shape[0]
    rp = -(-r // unit) * unit
    if rp != r:
        idx = jnp.pad(idx, (0, rp - r))
    out = _sc_gather_call(rp, d, rp // unit)(table, idx)
    return out[:r, :c]


def _gather_rows(src, idx):
    b, n, c = src.shape
    off = (jnp.arange(b, dtype=jnp.int32) * n).reshape(
        (b,) + (1,) * (idx.ndim - 1))
    flat = _sc_gather(src.reshape(b * n, c),
                      (idx.astype(jnp.int32) + off).reshape(-1))
    return flat.reshape(idx.shape + (c,))


def _xyz2sphere(xyz):
    rho = jnp.sqrt(jnp.sum(xyz ** 2, axis=-1, keepdims=True))
    rho_c = jnp.maximum(rho, _EPS)
    theta = jnp.arccos(jnp.clip(xyz[..., 2:3] / rho_c, -1.0, 1.0)) / np.pi
    phi = jnp.arctan2(xyz[..., 1:2], xyz[..., 0:1]) / (2 * np.pi) + 0.5
    return jnp.concatenate([rho, theta, phi], axis=-1)


def _index_points(points, idx):
    return jax.vmap(lambda p, i: p[i])(points, idx)


def _sq_dist(src, dst):
    return jnp.sum((src[:, :, None, :] - dst[:, None, :, :]) ** 2, axis=-1)


def _fps_body(xt_ref, o_ref, *, npoint):
    b, _, n = xt_ref.shape
    xt = xt_ref[...]
    iota = jax.lax.broadcasted_iota(jnp.int32, (b, n), 1)
    col = jax.lax.broadcasted_iota(jnp.int32, (b, npoint), 1)

    def step(i, carry):
        dists, idxs = carry
        m = jnp.max(dists, axis=1, keepdims=True)
        far = jnp.min(jnp.where(dists == m, iota, n), axis=1, keepdims=True)
        idxs = jnp.where(col == i, jnp.broadcast_to(far, (b, npoint)), idxs)
        oh = jnp.broadcast_to(far, (b, n)) == iota
        xf = jnp.sum(jnp.where(oh[:, None, :], xt, 0.0), axis=2)
        dx = xt[:, 0, :] - xf[:, 0:1]
        dy = xt[:, 1, :] - xf[:, 1:2]
        dz = xt[:, 2, :] - xf[:, 2:3]
        d = dx * dx + dy * dy + dz * dz
        dists = jnp.minimum(dists, d)
        return dists, idxs

    row = jax.lax.broadcasted_iota(jnp.int32, (b, npoint), 0)
    init = (jnp.full((b, n), 1e10, dtype=_F32),
            jnp.minimum(col + row, 0))
    o_ref[...] = jax.lax.fori_loop(0, npoint, step, init)[1]


def _fps(xyz, npoint):
    b, n, _ = xyz.shape
    xt = jnp.transpose(xyz, (0, 2, 1))
    return pl.pallas_call(
        lambda xt_ref, o_ref: _fps_body(xt_ref, o_ref, npoint=npoint),
        out_shape=jax.ShapeDtypeStruct((b, npoint), jnp.int32),
    )(xt)


def _pair_dists(q_ref, xt_ref):
    q = q_ref[0]
    dx = q[:, 0:1] - xt_ref[0, 0:1, :]
    dy = q[:, 1:2] - xt_ref[0, 1:2, :]
    dz = q[:, 2:3] - xt_ref[0, 2:3, :]
    return dx * dx + dy * dy + dz * dz


def _concrete_zero_i32(shape):
    a = jax.lax.broadcasted_iota(jnp.int32, shape, 0)
    b = jax.lax.broadcasted_iota(jnp.int32, shape, 1)
    return jnp.minimum(a + b, 0)


def _knn_body(q_ref, xt_ref, o_ref, *, k):
    qn, n = q_ref.shape[1], xt_ref.shape[2]
    d = _pair_dists(q_ref, xt_ref)
    iota = jax.lax.broadcasted_iota(jnp.int32, (qn, n), 1)
    colk = jax.lax.broadcasted_iota(jnp.int32, (qn, k), 1)
    out = _concrete_zero_i32((qn, k))

    def round_(i, carry):
        d, out = carry
        m = jnp.min(d, axis=1, keepdims=True)
        j = jnp.min(jnp.where(d == m, iota, n), axis=1, keepdims=True)
        out = jnp.where(colk == i, jnp.broadcast_to(j, (qn, k)), out)
        d = jnp.where(jnp.broadcast_to(j, (qn, n)) == iota, jnp.inf, d)
        return d, out

    o_ref[0] = jax.lax.fori_loop(0, k, round_, (d, out))[1]


def _knn(k, xyz):
    b, n, _ = xyz.shape
    xt = jnp.transpose(xyz, (0, 2, 1))
    qb = min(n, 256)
    return pl.pallas_call(
        lambda q_ref, xt_ref, o_ref: _knn_body(q_ref, xt_ref, o_ref, k=k),
        grid=(b, n // qb),
        in_specs=[pl.BlockSpec((1, qb, 3), lambda i, j: (i, j, 0)),
                  pl.BlockSpec((1, 3, n), lambda i, j: (i, 0, 0))],
        out_specs=pl.BlockSpec((1, qb, k), lambda i, j: (i, j, 0)),
        out_shape=jax.ShapeDtypeStruct((b, n, k), jnp.int32),
    )(xyz, xt)


def _ball_body(q_ref, xt_ref, o_ref, *, nsample, r2):
    qn, n = q_ref.shape[1], xt_ref.shape[2]
    d = _pair_dists(q_ref, xt_ref)
    iota = jax.lax.broadcasted_iota(jnp.int32, (qn, n), 1)
    colk = jax.lax.broadcasted_iota(jnp.int32, (qn, nsample), 1)
    key = jnp.where(d > r2, n, iota)
    out = _concrete_zero_i32((qn, nsample))

    def round_(i, carry):
        key, out = carry
        j = jnp.min(key, axis=1, keepdims=True)
        out = jnp.where(colk == i, jnp.broadcast_to(j, (qn, nsample)), out)
        key = jnp.where(jnp.broadcast_to(j, (qn, n)) == iota, n, key)
        return key, out

    out = jax.lax.fori_loop(0, nsample, round_, (key, out))[1]
    first = jnp.broadcast_to(out[:, 0:1], (qn, nsample))
    o_ref[0] = jnp.where(out == n, first, out)


def _ball_query(radius, nsample, xyz, new_xyz):
    b, n, _ = xyz.shape
    s = new_xyz.shape[1]
    xt = jnp.transpose(xyz, (0, 2, 1))
    qb = min(s, 512)
    return pl.pallas_call(
        lambda q_ref, xt_ref, o_ref: _ball_body(
            q_ref, xt_ref, o_ref, nsample=nsample, r2=radius ** 2),
        grid=(b, s // qb),
        in_specs=[pl.BlockSpec((1, qb, 3), lambda i, j: (i, j, 0)),
                  pl.BlockSpec((1, 3, n), lambda i, j: (i, 0, 0))],
        out_specs=pl.BlockSpec((1, qb, nsample), lambda i, j: (i, j, 0)),
        out_shape=jax.ShapeDtypeStruct((b, s, nsample), jnp.int32),
    )(new_xyz, xt)


def _umbrella_surface(center, p, k):
    b, n, _ = center.shape
    idx = _knn(k, center)
    gxyz = _gather_rows(center, idx[:, :, 1:])
    gnorm = gxyz - center[:, :, None, :]
    phi = _xyz2sphere(gnorm)[..., 2]
    kk1 = k - 1
    ar = jnp.arange(kk1)
    lt = phi[..., None, :] < phi[..., :, None]
    eq = (phi[..., None, :] == phi[..., :, None]) & (ar[None, :] < ar[:, None])
    rank = jnp.sum(lt | eq, axis=-1)
    perm = (rank[..., None, :] == ar[:, None]).astype(_F32)
    sg = jnp.matmul(perm, gnorm)[:, :, :, None, :]
    sgr = jnp.roll(sg, -1, axis=2)
    g = jnp.concatenate([jnp.zeros_like(sg), sg, sgr], axis=-2)

    e1 = g[..., 1, :] - g[..., 0, :]
    e2 = g[..., 2, :] - g[..., 0, :]
    nor = jnp.cross(e1, e2)
    unit = nor / jnp.maximum(
        jnp.linalg.norm(nor, axis=-1, keepdims=True), _EPS)
    pos_mask = (unit[..., 0:1, 0] > 0).astype(_F32) * 2.0 - 1.0
    unit = unit * pos_mask[..., None]
    gc = jnp.mean(g, axis=-2)
    gpolar = _xyz2sphere(gc)
    gpos = jnp.sum(unit * gc, axis=-1, keepdims=True) / np.sqrt(3.0)
    feat = jnp.concatenate([gc, gpolar, unit, gpos], axis=-1)
    feat = jnp.nan_to_num(feat)

    kk = k - 1
    rows = b * n * kk
    x = feat.reshape(rows, feat.shape[-1])
    y1, s1 = _mm_stats(x, p['W1'].T, p['b1'])
    a1, c1 = _bn_affine(s1, rows, p['g1'], p['bt1'])
    y2, _ = _mm_stats(y1, p['W2'].T, p['b2'], affine=(a1, c1))
    out = _pool(y2, jnp.ones_like(p['b2']), jnp.zeros_like(p['b2']),
                kk, relu=False, mode="sum")
    return out.reshape(b, n, -1)


def _sa_stage(center, normal, feature, p, npoint, radius, nsample,
              pos_channel, group_all):
    b = center.shape[0]
    if group_all:
        new_center = jnp.zeros((b, 1, 3), dtype=center.dtype)
        new_normal = jnp.zeros((b, 1, normal.shape[-1]), dtype=normal.dtype)
        parts = [center[:, None], normal[:, None]]
        if feature is not None:
            parts.append(feature[:, None])
        nf = jnp.concatenate(parts, axis=-1)
        k = center.shape[1]
        s = nf.shape[1]
        rows = b * s * k
        cin = nf.shape[-1]
        flat = nf.reshape(rows, cin)
        xp = flat[:, :pos_channel]
        xf = flat[:, pos_channel:]
    else:
        parts = [center, normal]
        if feature is not None:
            parts.append(feature)
        table = jnp.concatenate(parts, axis=-1)
        fps_idx = _fps(center, npoint)
        sampled = _gather_rows(table, fps_idx)
        new_center = sampled[..., :3]
        new_normal = sampled[..., 3:3 + normal.shape[-1]]
        idx = _ball_query(radius, nsample, center, new_center)
        nbr = _gather_rows(table, idx)
        k = nsample
        s = npoint
        rows = b * s * k
        cin = table.shape[-1]
        flat = nbr.reshape(rows, cin)
        xp = (flat[:, :pos_channel]
              - jnp.broadcast_to(new_center[:, :, None, :],
                                 (b, s, k, 3)).reshape(rows, 3))
        xf = flat[:, pos_channel:]

    yl, sl = _mm_stats(xp, p['Wl0'].T, p['bl0'])
    yf, sf = _mm_stats(xf, p['Wf0'].T, p['bf0'])
    al, cl = _bn_affine(sl, rows, p['gl0'], p['btl0'])
    af, cf = _bn_affine(sf, rows, p['gf0'], p['btf0'])

    y = None
    aff = None
    first = True
    for lay in p['layers']:
        if first:
            y, st = _mm_stats(yl, lay['W'].T, lay['b'],
                              affine=(al, cl), x2=yf, affine2=(af, cf))
            first = False
        else:
            y, st = _mm_stats(y, lay['W'].T, lay['b'], affine=aff)
        aff = _bn_affine(st, rows, lay['g'], lay['bt'])

    f = _pool(y, aff[0], aff[1], k, relu=True, mode="max")
    return new_center, new_normal, f.reshape(b, s, -1)


def kernel(points, params):
    center = jnp.transpose(points[:, :3, :], (0, 2, 1))
    normal = _umbrella_surface(center, params['umb'], 9)
    c, n, f = _sa_stage(center, normal, None, params['sa1'],
                        512, 0.1, 24, 3, False)
    c, n, f = _sa_stage(c, n, f, params['sa2'], 128, 0.2, 24, 3, False)
    c, n, f = _sa_stage(c, n, f, params['sa3'], 32, 0.4, 24, 3, False)
    c, n, f = _sa_stage(c, n, f, params['sa4'], None, None, None, 3, True)
    feat = f.reshape(-1, 2048)
    return _classifier(feat, params['cls'])


# speedup vs baseline: 7.1955x; 1.0665x over previous
import functools

import numpy as np
import jax
import jax.numpy as jnp
from jax.experimental import pallas as pl
from jax.experimental.pallas import tpu as pltpu
from jax.experimental.pallas import tpu_sc as plsc

_EPS = 1e-8
_F32 = jnp.float32


def _pick_br(rows, mult, cap):
    best = mult
    k = 1
    while True:
        cand = mult * (k + 1)
        if cand > cap or cand > rows:
            break
        k += 1
        if rows % cand == 0:
            best = cand
    return best if rows % best == 0 else rows


def _stats_update(s_ref, y):
    c = y.shape[-1]
    row0 = jnp.sum(y, axis=0, keepdims=True)
    row1 = jnp.sum(y * y, axis=0, keepdims=True)
    upd = jnp.concatenate(
        [row0, row1, jnp.zeros((6, c), dtype=y.dtype)], axis=0)

    @pl.when(pl.program_id(0) == 0)
    def _():
        s_ref[...] = jnp.zeros_like(s_ref)

    s_ref[...] += upd


def _mm0_body(x_ref, wt_ref, b_ref, y_ref, s_ref):
    y = jnp.dot(x_ref[...], wt_ref[...],
                preferred_element_type=_F32) + b_ref[...]
    y_ref[...] = y
    _stats_update(s_ref, y)


def _mm1_body(x_ref, a_ref, c_ref, wt_ref, b_ref, y_ref, s_ref):
    h = jnp.maximum(x_ref[...] * a_ref[...] + c_ref[...], 0.0)
    y = jnp.dot(h, wt_ref[...], preferred_element_type=_F32) + b_ref[...]
    y_ref[...] = y
    _stats_update(s_ref, y)


def _mm2_body(x1_ref, a1_ref, c1_ref, x2_ref, a2_ref, c2_ref,
              wt_ref, b_ref, y_ref, s_ref):
    pre = (x1_ref[...] * a1_ref[...] + c1_ref[...]
           + x2_ref[...] * a2_ref[...] + c2_ref[...])
    h = jnp.maximum(pre, 0.0)
    y = jnp.dot(h, wt_ref[...], preferred_element_type=_F32) + b_ref[...]
    y_ref[...] = y
    _stats_update(s_ref, y)


def _mm1_pool_body(x_ref, a_ref, c_ref, wt_ref, b_ref, o_ref, s_ref, *, k):
    h = jnp.maximum(x_ref[...] * a_ref[...] + c_ref[...], 0.0)
    y = jnp.dot(h, wt_ref[...], preferred_element_type=_F32) + b_ref[...]
    _stats_update(s_ref, y)
    br, cout = y.shape
    o_ref[...] = jnp.max(y.reshape(br // k, k, cout), axis=1)


def _sum_mm_body(x_ref, a_ref, c_ref, wt_ref, b_ref, o_ref, *, k):
    h = jnp.maximum(x_ref[...] * a_ref[...] + c_ref[...], 0.0)
    br, cin = h.shape
    hp = jnp.sum(h.reshape(br // k, k, cin), axis=1)
    o_ref[...] = jnp.dot(hp, wt_ref[...],
                         preferred_element_type=_F32) + b_ref[...]


def _affine_relu_body(x_ref, a_ref, c_ref, o_ref):
    o_ref[...] = jnp.maximum(x_ref[...] * a_ref[...] + c_ref[...], 0.0)


def _pool_body(x_ref, a_ref, c_ref, o_ref, *, relu, mode):
    h = x_ref[...] * a_ref[...] + c_ref[...]
    if relu:
        h = jnp.maximum(h, 0.0)
    if mode == "max":
        o_ref[...] = jnp.max(h, axis=1)
    else:
        o_ref[...] = jnp.sum(h, axis=1)


def _mm_stats(x, wt, b, affine=None, x2=None, affine2=None):
    rows, cin = x.shape
    cout = wt.shape[1]
    pcin = -(-cin // 128) * 128
    pcout = -(-cout // 128) * 128
    nin = 2 if x2 is not None else 1
    cap = max(8, (4 << 20) // (8 * (nin * pcin + pcout)))
    br = _pick_br(rows, 8, min(cap, rows))
    grid = (rows // br,)
    b2 = b.reshape(1, cout)
    out_shape = [jax.ShapeDtypeStruct((rows, cout), _F32),
                 jax.ShapeDtypeStruct((8, cout), _F32)]
    out_specs = [pl.BlockSpec((br, cout), lambda i: (i, 0)),
                 pl.BlockSpec((8, cout), lambda i: (0, 0))]
    w_spec = pl.BlockSpec((cin, cout), lambda i: (0, 0))
    bias_spec = pl.BlockSpec((1, cout), lambda i: (0, 0))
    row_spec = pl.BlockSpec((br, cin), lambda i: (i, 0))
    vec_spec = pl.BlockSpec((1, cin), lambda i: (0, 0))

    if affine is None:
        y, s = pl.pallas_call(
            _mm0_body, grid=grid,
            in_specs=[row_spec, w_spec, bias_spec],
            out_specs=out_specs, out_shape=out_shape,
        )(x, wt, b2)
    elif x2 is None:
        a, c = affine
        y, s = pl.pallas_call(
            _mm1_body, grid=grid,
            in_specs=[row_spec, vec_spec, vec_spec, w_spec, bias_spec],
            out_specs=out_specs, out_shape=out_shape,
        )(x, a.reshape(1, cin), c.reshape(1, cin), wt, b2)
    else:
        a1, c1 = affine
        a2, c2 = affine2
        y, s = pl.pallas_call(
            _mm2_body, grid=grid,
            in_specs=[row_spec, vec_spec, vec_spec,
                      row_spec, vec_spec, vec_spec, w_spec, bias_spec],
            out_specs=out_specs, out_shape=out_shape,
        )(x, a1.reshape(1, cin), c1.reshape(1, cin),
          x2, a2.reshape(1, cin), c2.reshape(1, cin), wt, b2)
    return y, s


def _mm_pool_stats(x, wt, b, affine, k):
    rows, cin = x.shape
    cout = wt.shape[1]
    pcin = -(-cin // 128) * 128
    pcout = -(-cout // 128) * 128
    cap = max(8 * k, (4 << 20) // (8 * (pcin + pcout)))
    br = _pick_br(rows, 8 * k, min(cap, rows))
    grid = (rows // br,)
    a, c = affine
    y, s = pl.pallas_call(
        lambda *refs: _mm1_pool_body(*refs, k=k),
        grid=grid,
        in_specs=[pl.BlockSpec((br, cin), lambda i: (i, 0)),
                  pl.BlockSpec((1, cin), lambda i: (0, 0)),
                  pl.BlockSpec((1, cin), lambda i: (0, 0)),
                  pl.BlockSpec((cin, cout), lambda i: (0, 0)),
                  pl.BlockSpec((1, cout), lambda i: (0, 0))],
        out_specs=[pl.BlockSpec((br // k, cout), lambda i: (i, 0)),
                   pl.BlockSpec((8, cout), lambda i: (0, 0))],
        out_shape=[jax.ShapeDtypeStruct((rows // k, cout), _F32),
                   jax.ShapeDtypeStruct((8, cout), _F32)],
    )(x, a.reshape(1, cin), c.reshape(1, cin), wt, b.reshape(1, cout))
    return y, s


def _sum_mm(x, wt, bsum, affine, k):
    rows, cin = x.shape
    cout = wt.shape[1]
    pcin = -(-cin // 128) * 128
    cap = max(8 * k, (4 << 20) // (8 * 2 * pcin))
    br = _pick_br(rows, 8 * k, min(cap, rows))
    a, c = affine
    return pl.pallas_call(
        lambda *refs: _sum_mm_body(*refs, k=k),
        grid=(rows // br,),
        in_specs=[pl.BlockSpec((br, cin), lambda i: (i, 0)),
                  pl.BlockSpec((1, cin), lambda i: (0, 0)),
                  pl.BlockSpec((1, cin), lambda i: (0, 0)),
                  pl.BlockSpec((cin, cout), lambda i: (0, 0)),
                  pl.BlockSpec((1, cout), lambda i: (0, 0))],
        out_specs=pl.BlockSpec((br // k, cout), lambda i: (i, 0)),
        out_shape=jax.ShapeDtypeStruct((rows // k, cout), _F32),
    )(x, a.reshape(1, cin), c.reshape(1, cin), wt, bsum.reshape(1, cout))


def _affine_relu(x, a, c):
    rows, cin = x.shape
    pcin = -(-cin // 128) * 128
    br = _pick_br(rows, 8, max(8, (2 << 20) // (8 * pcin)))
    return pl.pallas_call(
        _affine_relu_body,
        grid=(rows // br,),
        in_specs=[pl.BlockSpec((br, cin), lambda i: (i, 0)),
                  pl.BlockSpec((1, cin), lambda i: (0, 0)),
                  pl.BlockSpec((1, cin), lambda i: (0, 0))],
        out_specs=pl.BlockSpec((br, cin), lambda i: (i, 0)),
        out_shape=jax.ShapeDtypeStruct((rows, cin), _F32),
    )(x, a.reshape(1, cin), c.reshape(1, cin))


def _bn_affine(stats, rows, gamma, beta):
    mean = stats[0] / rows
    var = stats[1] / rows - mean * mean
    a = gamma / jnp.sqrt(var + 1e-5)
    c = beta - mean * a
    return a, c


def _pool(x, a, c, k, relu, mode):
    rows, cin = x.shape
    groups = rows // k
    x3 = x.reshape(groups, k, cin)
    pcin = -(-cin // 128) * 128
    gb = _pick_br(groups, 8, max(8, (1 << 21) // (4 * k * pcin)))
    if groups % gb != 0 or groups < 8:
        gb = groups
    y = pl.pallas_call(
        lambda x_ref, a_ref, c_ref, o_ref: _pool_body(
            x_ref, a_ref, c_ref, o_ref, relu=relu, mode=mode),
        grid=(groups // gb,),
        in_specs=[pl.BlockSpec((gb, k, cin), lambda i: (i, 0, 0)),
                  pl.BlockSpec((1, 1, cin), lambda i: (0, 0, 0)),
                  pl.BlockSpec((1, 1, cin), lambda i: (0, 0, 0))],
        out_specs=pl.BlockSpec((gb, cin), lambda i: (i, 0)),
        out_shape=jax.ShapeDtypeStruct((groups, cin), _F32),
    )(x3, a.reshape(1, 1, cin), c.reshape(1, 1, cin))
    return y


def _cls_body(x_ref, w1_ref, b1_ref, g1_ref, t1_ref,
              w2_ref, b2_ref, g2_ref, t2_ref,
              w3_ref, b3_ref, o_ref):
    def bn_relu(y, g, t):
        mean = jnp.mean(y, axis=0, keepdims=True)
        var = jnp.mean(y * y, axis=0, keepdims=True) - mean * mean
        return jnp.maximum(g * (y - mean) / jnp.sqrt(var + 1e-5) + t, 0.0)

    h = jnp.dot(x_ref[...], w1_ref[...], preferred_element_type=_F32) \
        + b1_ref[...]
    h = bn_relu(h, g1_ref[...], t1_ref[...])
    h = jnp.dot(h, w2_ref[...], preferred_element_type=_F32) + b2_ref[...]
    h = bn_relu(h, g2_ref[...], t2_ref[...])
    z = jnp.dot(h, w3_ref[...], preferred_element_type=_F32) + b3_ref[...]
    m = jnp.max(z, axis=-1, keepdims=True)
    lse = m + jnp.log(jnp.sum(jnp.exp(z - m), axis=-1, keepdims=True))
    o_ref[...] = z - lse


def _classifier(feat, cp):
    b = feat.shape[0]
    args = (feat,
            cp['W1'].T, cp['b1'].reshape(1, -1),
            cp['g1'].reshape(1, -1), cp['bt1'].reshape(1, -1),
            cp['W2'].T, cp['b2'].reshape(1, -1),
            cp['g2'].reshape(1, -1), cp['bt2'].reshape(1, -1),
            cp['W3'].T, cp['b3'].reshape(1, -1))
    return pl.pallas_call(
        _cls_body,
        out_shape=jax.ShapeDtypeStruct((b, cp['W3'].shape[0]), _F32),
    )(*args)


_SC_CH = 128
_SC_NC = 2
_SC_NS = 16
_SC_NW = _SC_NC * _SC_NS


def _sc_gather_call(rows, d, n_chunks):
    mesh = plsc.VectorSubcoreMesh(core_axis_name="c", subcore_axis_name="s")
    nc = _SC_NC
    b_per_w = n_chunks * _SC_CH

    @functools.partial(
        pl.kernel, mesh=mesh,
        out_type=jax.ShapeDtypeStruct((rows, d), jnp.float32),
        scratch_types=[pltpu.VMEM((_SC_CH,), jnp.int32),
                       pltpu.VMEM((_SC_CH, d), jnp.float32),
                       pltpu.SemaphoreType.DMA],
    )
    def gather_k(table_hbm, idx_hbm, out_hbm, idx_v, rows_v, sem):
        wid = jax.lax.axis_index("s") * nc + jax.lax.axis_index("c")
        base = wid * b_per_w

        def body(i, carry):
            off = base + i * _SC_CH
            pltpu.sync_copy(idx_hbm.at[pl.ds(off, _SC_CH)], idx_v)
            pltpu.async_copy(table_hbm.at[idx_v], rows_v, sem).wait()
            pltpu.sync_copy(rows_v, out_hbm.at[pl.ds(off, _SC_CH)])
            return carry

        jax.lax.fori_loop(0, n_chunks, body, 0)

    return gather_k


def _sc_gather(table, idx):
    t, c = table.shape
    d = -(-c // 128) * 128
    if d != c:
        table = jnp.pad(table, ((0, 0), (0, d - c)))
    unit = _SC_NW * _SC_CH
    r = idx.shape[0]
    rp = -(-r // unit) * unit
    if rp != r:
        idx = jnp.pad(idx, (0, rp - r))
    out = _sc_gather_call(rp, d, rp // unit)(table, idx)
    return out[:r, :c]


def _gather_rows(src, idx):
    b, n, c = src.shape
    off = (jnp.arange(b, dtype=jnp.int32) * n).reshape(
        (b,) + (1,) * (idx.ndim - 1))
    flat = _sc_gather(src.reshape(b * n, c),
                      (idx.astype(jnp.int32) + off).reshape(-1))
    return flat.reshape(idx.shape + (c,))


def _xyz2sphere(xyz):
    rho = jnp.sqrt(jnp.sum(xyz ** 2, axis=-1, keepdims=True))
    rho_c = jnp.maximum(rho, _EPS)
    theta = jnp.arccos(jnp.clip(xyz[..., 2:3] / rho_c, -1.0, 1.0)) / np.pi
    phi = jnp.arctan2(xyz[..., 1:2], xyz[..., 0:1]) / (2 * np.pi) + 0.5
    return jnp.concatenate([rho, theta, phi], axis=-1)


def _index_points(points, idx):
    return jax.vmap(lambda p, i: p[i])(points, idx)


def _sq_dist(src, dst):
    return jnp.sum((src[:, :, None, :] - dst[:, None, :, :]) ** 2, axis=-1)


def _fps_body(xt_ref, o_ref, *, npoint):
    b, _, n = xt_ref.shape
    xt = xt_ref[...]
    iota = jax.lax.broadcasted_iota(jnp.int32, (b, n), 1)
    col = jax.lax.broadcasted_iota(jnp.int32, (b, npoint), 1)

    def step(i, carry):
        dists, idxs = carry
        m = jnp.max(dists, axis=1, keepdims=True)
        far = jnp.min(jnp.where(dists == m, iota, n), axis=1, keepdims=True)
        idxs = jnp.where(col == i, jnp.broadcast_to(far, (b, npoint)), idxs)
        oh = jnp.broadcast_to(far, (b, n)) == iota
        xf = jnp.sum(jnp.where(oh[:, None, :], xt, 0.0), axis=2)
        dx = xt[:, 0, :] - xf[:, 0:1]
        dy = xt[:, 1, :] - xf[:, 1:2]
        dz = xt[:, 2, :] - xf[:, 2:3]
        d = dx * dx + dy * dy + dz * dz
        dists = jnp.minimum(dists, d)
        return dists, idxs

    row = jax.lax.broadcasted_iota(jnp.int32, (b, npoint), 0)
    init = (jnp.full((b, n), 1e10, dtype=_F32),
            jnp.minimum(col + row, 0))
    o_ref[...] = jax.lax.fori_loop(0, npoint, step, init)[1]


def _fps(xyz, npoint):
    b, n, _ = xyz.shape
    xt = jnp.transpose(xyz, (0, 2, 1))
    return pl.pallas_call(
        lambda xt_ref, o_ref: _fps_body(xt_ref, o_ref, npoint=npoint),
        out_shape=jax.ShapeDtypeStruct((b, npoint), jnp.int32),
    )(xt)


def _pair_dists(q_ref, xt_ref):
    q = q_ref[0]
    dx = q[:, 0:1] - xt_ref[0, 0:1, :]
    dy = q[:, 1:2] - xt_ref[0, 1:2, :]
    dz = q[:, 2:3] - xt_ref[0, 2:3, :]
    return dx * dx + dy * dy + dz * dz


def _concrete_zero_i32(shape):
    a = jax.lax.broadcasted_iota(jnp.int32, shape, 0)
    b = jax.lax.broadcasted_iota(jnp.int32, shape, 1)
    return jnp.minimum(a + b, 0)


def _knn_body(q_ref, xt_ref, o_ref, *, k):
    qn, n = q_ref.shape[1], xt_ref.shape[2]
    d = _pair_dists(q_ref, xt_ref)
    iota = jax.lax.broadcasted_iota(jnp.int32, (qn, n), 1)
    colk = jax.lax.broadcasted_iota(jnp.int32, (qn, k), 1)
    out = _concrete_zero_i32((qn, k))

    def round_(i, carry):
        d, out = carry
        m = jnp.min(d, axis=1, keepdims=True)
        j = jnp.min(jnp.where(d == m, iota, n), axis=1, keepdims=True)
        out = jnp.where(colk == i, jnp.broadcast_to(j, (qn, k)), out)
        d = jnp.where(jnp.broadcast_to(j, (qn, n)) == iota, jnp.inf, d)
        return d, out

    o_ref[0] = jax.lax.fori_loop(0, k, round_, (d, out))[1]


def _knn(k, xyz):
    b, n, _ = xyz.shape
    xt = jnp.transpose(xyz, (0, 2, 1))
    qb = min(n, 256)
    return pl.pallas_call(
        lambda q_ref, xt_ref, o_ref: _knn_body(q_ref, xt_ref, o_ref, k=k),
        grid=(b, n // qb),
        in_specs=[pl.BlockSpec((1, qb, 3), lambda i, j: (i, j, 0)),
                  pl.BlockSpec((1, 3, n), lambda i, j: (i, 0, 0))],
        out_specs=pl.BlockSpec((1, qb, k), lambda i, j: (i, j, 0)),
        out_shape=jax.ShapeDtypeStruct((b, n, k), jnp.int32),
    )(xyz, xt)


def _ball_body(q_ref, xt_ref, o_ref, *, nsample, r2):
    qn, n = q_ref.shape[1], xt_ref.shape[2]
    d = _pair_dists(q_ref, xt_ref)
    iota = jax.lax.broadcasted_iota(jnp.int32, (qn, n), 1)
    colk = jax.lax.broadcasted_iota(jnp.int32, (qn, nsample), 1)
    key = jnp.where(d > r2, n, iota)
    out = _concrete_zero_i32((qn, nsample))

    def round_(i, carry):
        key, out = carry
        j = jnp.min(key, axis=1, keepdims=True)
        out = jnp.where(colk == i, jnp.broadcast_to(j, (qn, nsample)), out)
        key = jnp.where(jnp.broadcast_to(j, (qn, n)) == iota, n, key)
        return key, out

    out = jax.lax.fori_loop(0, nsample, round_, (key, out))[1]
    first = jnp.broadcast_to(out[:, 0:1], (qn, nsample))
    o_ref[0] = jnp.where(out == n, first, out)


def _ball_query(radius, nsample, xyz, new_xyz):
    b, n, _ = xyz.shape
    s = new_xyz.shape[1]
    xt = jnp.transpose(xyz, (0, 2, 1))
    qb = min(s, 512)
    return pl.pallas_call(
        lambda q_ref, xt_ref, o_ref: _ball_body(
            q_ref, xt_ref, o_ref, nsample=nsample, r2=radius ** 2),
        grid=(b, s // qb),
        in_specs=[pl.BlockSpec((1, qb, 3), lambda i, j: (i, j, 0)),
                  pl.BlockSpec((1, 3, n), lambda i, j: (i, 0, 0))],
        out_specs=pl.BlockSpec((1, qb, nsample), lambda i, j: (i, j, 0)),
        out_shape=jax.ShapeDtypeStruct((b, s, nsample), jnp.int32),
    )(new_xyz, xt)


def _umbrella_surface(center, p, k):
    b, n, _ = center.shape
    idx = _knn(k, center)
    gxyz = _gather_rows(center, idx[:, :, 1:])
    gnorm = gxyz - center[:, :, None, :]
    phi = _xyz2sphere(gnorm)[..., 2]
    kk1 = k - 1
    ar = jnp.arange(kk1)
    lt = phi[..., None, :] < phi[..., :, None]
    eq = (phi[..., None, :] == phi[..., :, None]) & (ar[None, :] < ar[:, None])
    rank = jnp.sum(lt | eq, axis=-1)
    perm = (rank[..., None, :] == ar[:, None]).astype(_F32)
    sg = jnp.matmul(perm, gnorm)[:, :, :, None, :]
    sgr = jnp.roll(sg, -1, axis=2)
    g = jnp.concatenate([jnp.zeros_like(sg), sg, sgr], axis=-2)

    e1 = g[..., 1, :] - g[..., 0, :]
    e2 = g[..., 2, :] - g[..., 0, :]
    nor = jnp.cross(e1, e2)
    unit = nor / jnp.maximum(
        jnp.linalg.norm(nor, axis=-1, keepdims=True), _EPS)
    pos_mask = (unit[..., 0:1, 0] > 0).astype(_F32) * 2.0 - 1.0
    unit = unit * pos_mask[..., None]
    gc = jnp.mean(g, axis=-2)
    gpolar = _xyz2sphere(gc)
    gpos = jnp.sum(unit * gc, axis=-1, keepdims=True) / np.sqrt(3.0)
    feat = jnp.concatenate([gc, gpolar, unit, gpos], axis=-1)
    feat = jnp.nan_to_num(feat)

    kk = k - 1
    rows = b * n * kk
    x = feat.reshape(rows, feat.shape[-1])
    y1, s1 = _mm_stats(x, p['W1'].T, p['b1'])
    a1, c1 = _bn_affine(s1, rows, p['g1'], p['bt1'])
    out = _sum_mm(y1, p['W2'].T, p['b2'] * kk, (a1, c1), kk)
    return out.reshape(b, n, -1)


def _sa_stage(center, normal, feature, p, npoint, radius, nsample,
              pos_channel, group_all):
    b = center.shape[0]
    if group_all:
        new_center = jnp.zeros((b, 1, 3), dtype=center.dtype)
        new_normal = jnp.zeros((b, 1, normal.shape[-1]), dtype=normal.dtype)
        parts = [center[:, None], normal[:, None]]
        if feature is not None:
            parts.append(feature[:, None])
        nf = jnp.concatenate(parts, axis=-1)
        k = center.shape[1]
        s = nf.shape[1]
        rows = b * s * k
        cin = nf.shape[-1]
        flat = nf.reshape(rows, cin)
        xp = flat[:, :pos_channel]
        xf = flat[:, pos_channel:]
    else:
        parts = [center, normal]
        if feature is not None:
            parts.append(feature)
        table = jnp.concatenate(parts, axis=-1)
        fps_idx = _fps(center, npoint)
        sampled = _gather_rows(table, fps_idx)
        new_center = sampled[..., :3]
        new_normal = sampled[..., 3:3 + normal.shape[-1]]
        idx = _ball_query(radius, nsample, center, new_center)
        nbr = _gather_rows(table, idx)
        k = nsample
        s = npoint
        rows = b * s * k
        cin = table.shape[-1]
        flat = nbr.reshape(rows, cin)
        xp = (flat[:, :pos_channel]
              - jnp.broadcast_to(new_center[:, :, None, :],
                                 (b, s, k, 3)).reshape(rows, 3))
        xf = flat[:, pos_channel:]

    yl, sl = _mm_stats(xp, p['Wl0'].T, p['bl0'])
    yf, sf = _mm_stats(xf, p['Wf0'].T, p['bf0'])
    al, cl = _bn_affine(sl, rows, p['gl0'], p['btl0'])
    af, cf = _bn_affine(sf, rows, p['gf0'], p['btf0'])

    lays = p['layers']
    y, st = _mm_stats(yl, lays[0]['W'].T, lays[0]['b'],
                      affine=(al, cl), x2=yf, affine2=(af, cf))
    aff = _bn_affine(st, rows, lays[0]['g'], lays[0]['bt'])
    for lay in lays[1:-1]:
        y, st = _mm_stats(y, lay['W'].T, lay['b'], affine=aff)
        aff = _bn_affine(st, rows, lay['g'], lay['bt'])
    y, st = _mm_pool_stats(y, lays[-1]['W'].T, lays[-1]['b'], aff, k)
    aff = _bn_affine(st, rows, lays[-1]['g'], lays[-1]['bt'])
    f = _affine_relu(y, aff[0], aff[1])
    return new_center, new_normal, f.reshape(b, s, -1)


def kernel(points, params):
    center = jnp.transpose(points[:, :3, :], (0, 2, 1))
    normal = _umbrella_surface(center, params['umb'], 9)
    c, n, f = _sa_stage(center, normal, None, params['sa1'],
                        512, 0.1, 24, 3, False)
    c, n, f = _sa_stage(c, n, f, params['sa2'], 128, 0.2, 24, 3, False)
    c, n, f = _sa_stage(c, n, f, params['sa3'], 32, 0.4, 24, 3, False)
    c, n, f = _sa_stage(c, n, f, params['sa4'], None, None, None, 3, True)
    feat = f.reshape(-1, 2048)
    return _classifier(feat, params['cls'])


# speedup vs baseline: 7.2714x; 1.0105x over previous
import functools

import numpy as np
import jax
import jax.numpy as jnp
from jax.experimental import pallas as pl
from jax.experimental.pallas import tpu as pltpu
from jax.experimental.pallas import tpu_sc as plsc

_EPS = 1e-8
_F32 = jnp.float32


def _pick_br(rows, mult, cap):
    best = mult
    k = 1
    while True:
        cand = mult * (k + 1)
        if cand > cap or cand > rows:
            break
        k += 1
        if rows % cand == 0:
            best = cand
    return best if rows % best == 0 else rows


def _stats_update(s_ref, y):
    c = y.shape[-1]
    row0 = jnp.sum(y, axis=0, keepdims=True)
    row1 = jnp.sum(y * y, axis=0, keepdims=True)
    upd = jnp.concatenate(
        [row0, row1, jnp.zeros((6, c), dtype=y.dtype)], axis=0)

    @pl.when(pl.program_id(0) == 0)
    def _():
        s_ref[...] = jnp.zeros_like(s_ref)

    s_ref[...] += upd


def _mm0_body(x_ref, wt_ref, b_ref, y_ref, s_ref):
    y = jnp.dot(x_ref[...], wt_ref[...],
                preferred_element_type=_F32) + b_ref[...]
    y_ref[...] = y
    _stats_update(s_ref, y)


def _mm1_body(x_ref, a_ref, c_ref, wt_ref, b_ref, y_ref, s_ref):
    h = jnp.maximum(x_ref[...] * a_ref[...] + c_ref[...], 0.0)
    y = jnp.dot(h, wt_ref[...], preferred_element_type=_F32) + b_ref[...]
    y_ref[...] = y
    _stats_update(s_ref, y)


def _mm2_body(x1_ref, a1_ref, c1_ref, x2_ref, a2_ref, c2_ref,
              wt_ref, b_ref, y_ref, s_ref):
    pre = (x1_ref[...] * a1_ref[...] + c1_ref[...]
           + x2_ref[...] * a2_ref[...] + c2_ref[...])
    h = jnp.maximum(pre, 0.0)
    y = jnp.dot(h, wt_ref[...], preferred_element_type=_F32) + b_ref[...]
    y_ref[...] = y
    _stats_update(s_ref, y)


def _mm1_pool_body(x_ref, a_ref, c_ref, wt_ref, b_ref, o_ref, s_ref, *, k):
    h = jnp.maximum(x_ref[...] * a_ref[...] + c_ref[...], 0.0)
    y = jnp.dot(h, wt_ref[...], preferred_element_type=_F32) + b_ref[...]
    _stats_update(s_ref, y)
    br, cout = y.shape
    o_ref[...] = jnp.max(y.reshape(br // k, k, cout), axis=1)


def _sum_mm_body(x_ref, a_ref, c_ref, wt_ref, b_ref, o_ref, *, k):
    h = jnp.maximum(x_ref[...] * a_ref[...] + c_ref[...], 0.0)
    br, cin = h.shape
    hp = jnp.sum(h.reshape(br // k, k, cin), axis=1)
    o_ref[...] = jnp.dot(hp, wt_ref[...],
                         preferred_element_type=_F32) + b_ref[...]


def _affine_relu_body(x_ref, a_ref, c_ref, o_ref):
    o_ref[...] = jnp.maximum(x_ref[...] * a_ref[...] + c_ref[...], 0.0)


def _pool_body(x_ref, a_ref, c_ref, o_ref, *, relu, mode):
    h = x_ref[...] * a_ref[...] + c_ref[...]
    if relu:
        h = jnp.maximum(h, 0.0)
    if mode == "max":
        o_ref[...] = jnp.max(h, axis=1)
    else:
        o_ref[...] = jnp.sum(h, axis=1)


def _mm_stats(x, wt, b, affine=None, x2=None, affine2=None):
    rows, cin = x.shape
    cout = wt.shape[1]
    pcin = -(-cin // 128) * 128
    pcout = -(-cout // 128) * 128
    nin = 2 if x2 is not None else 1
    cap = max(8, (4 << 20) // (8 * (nin * pcin + pcout)))
    br = _pick_br(rows, 8, min(cap, rows))
    grid = (rows // br,)
    b2 = b.reshape(1, cout)
    out_shape = [jax.ShapeDtypeStruct((rows, cout), _F32),
                 jax.ShapeDtypeStruct((8, cout), _F32)]
    out_specs = [pl.BlockSpec((br, cout), lambda i: (i, 0)),
                 pl.BlockSpec((8, cout), lambda i: (0, 0))]
    w_spec = pl.BlockSpec((cin, cout), lambda i: (0, 0))
    bias_spec = pl.BlockSpec((1, cout), lambda i: (0, 0))
    row_spec = pl.BlockSpec((br, cin), lambda i: (i, 0))
    vec_spec = pl.BlockSpec((1, cin), lambda i: (0, 0))

    if affine is None:
        y, s = pl.pallas_call(
            _mm0_body, grid=grid,
            in_specs=[row_spec, w_spec, bias_spec],
            out_specs=out_specs, out_shape=out_shape,
        )(x, wt, b2)
    elif x2 is None:
        a, c = affine
        y, s = pl.pallas_call(
            _mm1_body, grid=grid,
            in_specs=[row_spec, vec_spec, vec_spec, w_spec, bias_spec],
            out_specs=out_specs, out_shape=out_shape,
        )(x, a.reshape(1, cin), c.reshape(1, cin), wt, b2)
    else:
        a1, c1 = affine
        a2, c2 = affine2
        y, s = pl.pallas_call(
            _mm2_body, grid=grid,
            in_specs=[row_spec, vec_spec, vec_spec,
                      row_spec, vec_spec, vec_spec, w_spec, bias_spec],
            out_specs=out_specs, out_shape=out_shape,
        )(x, a1.reshape(1, cin), c1.reshape(1, cin),
          x2, a2.reshape(1, cin), c2.reshape(1, cin), wt, b2)
    return y, s


def _l0_stats_body(xp_ref, xf_ref, wl_ref, bl_ref, wf_ref, bf_ref,
                   sl_ref, sf_ref):
    yl = jnp.dot(xp_ref[...], wl_ref[...],
                 preferred_element_type=_F32) + bl_ref[...]
    _stats_update(sl_ref, yl)
    yf = jnp.dot(xf_ref[...], wf_ref[...],
                 preferred_element_type=_F32) + bf_ref[...]
    _stats_update(sf_ref, yf)


def _l0_mm2_body(xp_ref, xf_ref, wl_ref, bl_ref, wf_ref, bf_ref,
                 al_ref, cl_ref, af_ref, cf_ref, wt_ref, b_ref,
                 y_ref, s_ref):
    yl = jnp.dot(xp_ref[...], wl_ref[...],
                 preferred_element_type=_F32) + bl_ref[...]
    yf = jnp.dot(xf_ref[...], wf_ref[...],
                 preferred_element_type=_F32) + bf_ref[...]
    h = jnp.maximum(yl * al_ref[...] + cl_ref[...]
                    + yf * af_ref[...] + cf_ref[...], 0.0)
    y = jnp.dot(h, wt_ref[...], preferred_element_type=_F32) + b_ref[...]
    y_ref[...] = y
    _stats_update(s_ref, y)


def _l0_specs(br, cp, cf, c0):
    return [pl.BlockSpec((br, cp), lambda i: (i, 0)),
            pl.BlockSpec((br, cf), lambda i: (i, 0)),
            pl.BlockSpec((cp, c0), lambda i: (0, 0)),
            pl.BlockSpec((1, c0), lambda i: (0, 0)),
            pl.BlockSpec((cf, c0), lambda i: (0, 0)),
            pl.BlockSpec((1, c0), lambda i: (0, 0))]


def _l0_br(rows, cp, cf, c0, cout):
    pcf = -(-cf // 128) * 128
    pcout = -(-max(c0, cout) // 128) * 128
    cap = max(8, (4 << 20) // (8 * (128 + pcf + 2 * pcout)))
    return _pick_br(rows, 8, min(cap, rows))


def _l0_stats(xp, xf, wlt, bl, wft, bf):
    rows, cp = xp.shape
    cf = xf.shape[1]
    c0 = wlt.shape[1]
    br = _l0_br(rows, cp, cf, c0, c0)
    sl, sf = pl.pallas_call(
        _l0_stats_body, grid=(rows // br,),
        in_specs=_l0_specs(br, cp, cf, c0),
        out_specs=[pl.BlockSpec((8, c0), lambda i: (0, 0)),
                   pl.BlockSpec((8, c0), lambda i: (0, 0))],
        out_shape=[jax.ShapeDtypeStruct((8, c0), _F32),
                   jax.ShapeDtypeStruct((8, c0), _F32)],
    )(xp, xf, wlt, bl.reshape(1, c0), wft, bf.reshape(1, c0))
    return sl, sf


def _l0_mm2(xp, xf, wlt, bl, wft, bf, aff_l, aff_f, wt, b):
    rows, cp = xp.shape
    cf = xf.shape[1]
    c0 = wlt.shape[1]
    cout = wt.shape[1]
    br = _l0_br(rows, cp, cf, c0, cout)
    v = lambda x: x.reshape(1, -1)
    y, s = pl.pallas_call(
        _l0_mm2_body, grid=(rows // br,),
        in_specs=_l0_specs(br, cp, cf, c0) + [
            pl.BlockSpec((1, c0), lambda i: (0, 0)),
            pl.BlockSpec((1, c0), lambda i: (0, 0)),
            pl.BlockSpec((1, c0), lambda i: (0, 0)),
            pl.BlockSpec((1, c0), lambda i: (0, 0)),
            pl.BlockSpec((c0, cout), lambda i: (0, 0)),
            pl.BlockSpec((1, cout), lambda i: (0, 0))],
        out_specs=[pl.BlockSpec((br, cout), lambda i: (i, 0)),
                   pl.BlockSpec((8, cout), lambda i: (0, 0))],
        out_shape=[jax.ShapeDtypeStruct((rows, cout), _F32),
                   jax.ShapeDtypeStruct((8, cout), _F32)],
    )(xp, xf, wlt, v(bl), wft, v(bf),
      v(aff_l[0]), v(aff_l[1]), v(aff_f[0]), v(aff_f[1]), wt, v(b))
    return y, s


def _mm_pool_stats(x, wt, b, affine, k):
    rows, cin = x.shape
    cout = wt.shape[1]
    pcin = -(-cin // 128) * 128
    pcout = -(-cout // 128) * 128
    cap = max(8 * k, (4 << 20) // (8 * (pcin + pcout)))
    br = _pick_br(rows, 8 * k, min(cap, rows))
    grid = (rows // br,)
    a, c = affine
    y, s = pl.pallas_call(
        lambda *refs: _mm1_pool_body(*refs, k=k),
        grid=grid,
        in_specs=[pl.BlockSpec((br, cin), lambda i: (i, 0)),
                  pl.BlockSpec((1, cin), lambda i: (0, 0)),
                  pl.BlockSpec((1, cin), lambda i: (0, 0)),
                  pl.BlockSpec((cin, cout), lambda i: (0, 0)),
                  pl.BlockSpec((1, cout), lambda i: (0, 0))],
        out_specs=[pl.BlockSpec((br // k, cout), lambda i: (i, 0)),
                   pl.BlockSpec((8, cout), lambda i: (0, 0))],
        out_shape=[jax.ShapeDtypeStruct((rows // k, cout), _F32),
                   jax.ShapeDtypeStruct((8, cout), _F32)],
    )(x, a.reshape(1, cin), c.reshape(1, cin), wt, b.reshape(1, cout))
    return y, s


def _sum_mm(x, wt, bsum, affine, k):
    rows, cin = x.shape
    cout = wt.shape[1]
    pcin = -(-cin // 128) * 128
    cap = max(8 * k, (4 << 20) // (8 * 2 * pcin))
    br = _pick_br(rows, 8 * k, min(cap, rows))
    a, c = affine
    return pl.pallas_call(
        lambda *refs: _sum_mm_body(*refs, k=k),
        grid=(rows // br,),
        in_specs=[pl.BlockSpec((br, cin), lambda i: (i, 0)),
                  pl.BlockSpec((1, cin), lambda i: (0, 0)),
                  pl.BlockSpec((1, cin), lambda i: (0, 0)),
                  pl.BlockSpec((cin, cout), lambda i: (0, 0)),
                  pl.BlockSpec((1, cout), lambda i: (0, 0))],
        out_specs=pl.BlockSpec((br // k, cout), lambda i: (i, 0)),
        out_shape=jax.ShapeDtypeStruct((rows // k, cout), _F32),
    )(x, a.reshape(1, cin), c.reshape(1, cin), wt, bsum.reshape(1, cout))


def _affine_relu(x, a, c):
    rows, cin = x.shape
    pcin = -(-cin // 128) * 128
    br = _pick_br(rows, 8, max(8, (2 << 20) // (8 * pcin)))
    return pl.pallas_call(
        _affine_relu_body,
        grid=(rows // br,),
        in_specs=[pl.BlockSpec((br, cin), lambda i: (i, 0)),
                  pl.BlockSpec((1, cin), lambda i: (0, 0)),
                  pl.BlockSpec((1, cin), lambda i: (0, 0))],
        out_specs=pl.BlockSpec((br, cin), lambda i: (i, 0)),
        out_shape=jax.ShapeDtypeStruct((rows, cin), _F32),
    )(x, a.reshape(1, cin), c.reshape(1, cin))


def _bn_affine(stats, rows, gamma, beta):
    mean = stats[0] / rows
    var = stats[1] / rows - mean * mean
    a = gamma / jnp.sqrt(var + 1e-5)
    c = beta - mean * a
    return a, c


def _pool(x, a, c, k, relu, mode):
    rows, cin = x.shape
    groups = rows // k
    x3 = x.reshape(groups, k, cin)
    pcin = -(-cin // 128) * 128
    gb = _pick_br(groups, 8, max(8, (1 << 21) // (4 * k * pcin)))
    if groups % gb != 0 or groups < 8:
        gb = groups
    y = pl.pallas_call(
        lambda x_ref, a_ref, c_ref, o_ref: _pool_body(
            x_ref, a_ref, c_ref, o_ref, relu=relu, mode=mode),
        grid=(groups // gb,),
        in_specs=[pl.BlockSpec((gb, k, cin), lambda i: (i, 0, 0)),
                  pl.BlockSpec((1, 1, cin), lambda i: (0, 0, 0)),
                  pl.BlockSpec((1, 1, cin), lambda i: (0, 0, 0))],
        out_specs=pl.BlockSpec((gb, cin), lambda i: (i, 0)),
        out_shape=jax.ShapeDtypeStruct((groups, cin), _F32),
    )(x3, a.reshape(1, 1, cin), c.reshape(1, 1, cin))
    return y


def _cls_body(x_ref, w1_ref, b1_ref, g1_ref, t1_ref,
              w2_ref, b2_ref, g2_ref, t2_ref,
              w3_ref, b3_ref, o_ref):
    def bn_relu(y, g, t):
        mean = jnp.mean(y, axis=0, keepdims=True)
        var = jnp.mean(y * y, axis=0, keepdims=True) - mean * mean
        return jnp.maximum(g * (y - mean) / jnp.sqrt(var + 1e-5) + t, 0.0)

    h = jnp.dot(x_ref[...], w1_ref[...], preferred_element_type=_F32) \
        + b1_ref[...]
    h = bn_relu(h, g1_ref[...], t1_ref[...])
    h = jnp.dot(h, w2_ref[...], preferred_element_type=_F32) + b2_ref[...]
    h = bn_relu(h, g2_ref[...], t2_ref[...])
    z = jnp.dot(h, w3_ref[...], preferred_element_type=_F32) + b3_ref[...]
    m = jnp.max(z, axis=-1, keepdims=True)
    lse = m + jnp.log(jnp.sum(jnp.exp(z - m), axis=-1, keepdims=True))
    o_ref[...] = z - lse


def _classifier(feat, cp):
    b = feat.shape[0]
    args = (feat,
            cp['W1'].T, cp['b1'].reshape(1, -1),
            cp['g1'].reshape(1, -1), cp['bt1'].reshape(1, -1),
            cp['W2'].T, cp['b2'].reshape(1, -1),
            cp['g2'].reshape(1, -1), cp['bt2'].reshape(1, -1),
            cp['W3'].T, cp['b3'].reshape(1, -1))
    return pl.pallas_call(
        _cls_body,
        out_shape=jax.ShapeDtypeStruct((b, cp['W3'].shape[0]), _F32),
    )(*args)


_SC_CH = 128
_SC_NC = 2
_SC_NS = 16
_SC_NW = _SC_NC * _SC_NS


def _sc_gather_call(rows, d, n_chunks):
    mesh = plsc.VectorSubcoreMesh(core_axis_name="c", subcore_axis_name="s")
    nc = _SC_NC
    b_per_w = n_chunks * _SC_CH

    @functools.partial(
        pl.kernel, mesh=mesh,
        out_type=jax.ShapeDtypeStruct((rows, d), jnp.float32),
        scratch_types=[pltpu.VMEM((_SC_CH,), jnp.int32),
                       pltpu.VMEM((_SC_CH, d), jnp.float32),
                       pltpu.SemaphoreType.DMA],
    )
    def gather_k(table_hbm, idx_hbm, out_hbm, idx_v, rows_v, sem):
        wid = jax.lax.axis_index("s") * nc + jax.lax.axis_index("c")
        base = wid * b_per_w

        def body(i, carry):
            off = base + i * _SC_CH
            pltpu.sync_copy(idx_hbm.at[pl.ds(off, _SC_CH)], idx_v)
            pltpu.async_copy(table_hbm.at[idx_v], rows_v, sem).wait()
            pltpu.sync_copy(rows_v, out_hbm.at[pl.ds(off, _SC_CH)])
            return carry

        jax.lax.fori_loop(0, n_chunks, body, 0)

    return gather_k


def _sc_gather(table, idx):
    t, c = table.shape
    d = -(-c // 128) * 128
    if d != c:
        table = jnp.pad(table, ((0, 0), (0, d - c)))
    unit = _SC_NW * _SC_CH
    r = idx.shape[0]
    rp = -(-r // unit) * unit
    if rp != r:
        idx = jnp.pad(idx, (0, rp - r))
    out = _sc_gather_call(rp, d, rp // unit)(table, idx)
    return out[:r, :c]


def _gather_rows(src, idx):
    b, n, c = src.shape
    off = (jnp.arange(b, dtype=jnp.int32) * n).reshape(
        (b,) + (1,) * (idx.ndim - 1))
    flat = _sc_gather(src.reshape(b * n, c),
                      (idx.astype(jnp.int32) + off).reshape(-1))
    return flat.reshape(idx.shape + (c,))


def _xyz2sphere(xyz):
    rho = jnp.sqrt(jnp.sum(xyz ** 2, axis=-1, keepdims=True))
    rho_c = jnp.maximum(rho, _EPS)
    theta = jnp.arccos(jnp.clip(xyz[..., 2:3] / rho_c, -1.0, 1.0)) / np.pi
    phi = jnp.arctan2(xyz[..., 1:2], xyz[..., 0:1]) / (2 * np.pi) + 0.5
    return jnp.concatenate([rho, theta, phi], axis=-1)


def _index_points(points, idx):
    return jax.vmap(lambda p, i: p[i])(points, idx)


def _sq_dist(src, dst):
    return jnp.sum((src[:, :, None, :] - dst[:, None, :, :]) ** 2, axis=-1)


def _fps_body(xt_ref, o_ref, *, npoint):
    b, _, n = xt_ref.shape
    xt = xt_ref[...]
    iota = jax.lax.broadcasted_iota(jnp.int32, (b, n), 1)
    col = jax.lax.broadcasted_iota(jnp.int32, (b, npoint), 1)

    def step(i, carry):
        dists, idxs = carry
        m = jnp.max(dists, axis=1, keepdims=True)
        far = jnp.min(jnp.where(dists == m, iota, n), axis=1, keepdims=True)
        idxs = jnp.where(col == i, jnp.broadcast_to(far, (b, npoint)), idxs)
        oh = jnp.broadcast_to(far, (b, n)) == iota
        xf = jnp.sum(jnp.where(oh[:, None, :], xt, 0.0), axis=2)
        dx = xt[:, 0, :] - xf[:, 0:1]
        dy = xt[:, 1, :] - xf[:, 1:2]
        dz = xt[:, 2, :] - xf[:, 2:3]
        d = dx * dx + dy * dy + dz * dz
        dists = jnp.minimum(dists, d)
        return dists, idxs

    row = jax.lax.broadcasted_iota(jnp.int32, (b, npoint), 0)
    init = (jnp.full((b, n), 1e10, dtype=_F32),
            jnp.minimum(col + row, 0))
    o_ref[...] = jax.lax.fori_loop(0, npoint, step, init)[1]


def _fps(xyz, npoint):
    b, n, _ = xyz.shape
    xt = jnp.transpose(xyz, (0, 2, 1))
    return pl.pallas_call(
        lambda xt_ref, o_ref: _fps_body(xt_ref, o_ref, npoint=npoint),
        out_shape=jax.ShapeDtypeStruct((b, npoint), jnp.int32),
    )(xt)


def _pair_dists(q_ref, xt_ref):
    q = q_ref[0]
    dx = q[:, 0:1] - xt_ref[0, 0:1, :]
    dy = q[:, 1:2] - xt_ref[0, 1:2, :]
    dz = q[:, 2:3] - xt_ref[0, 2:3, :]
    return dx * dx + dy * dy + dz * dz


def _concrete_zero_i32(shape):
    a = jax.lax.broadcasted_iota(jnp.int32, shape, 0)
    b = jax.lax.broadcasted_iota(jnp.int32, shape, 1)
    return jnp.minimum(a + b, 0)


def _knn_body(q_ref, xt_ref, o_ref, *, k):
    qn, n = q_ref.shape[1], xt_ref.shape[2]
    d = _pair_dists(q_ref, xt_ref)
    iota = jax.lax.broadcasted_iota(jnp.int32, (qn, n), 1)
    colk = jax.lax.broadcasted_iota(jnp.int32, (qn, k), 1)
    out = _concrete_zero_i32((qn, k))

    def round_(i, carry):
        d, out = carry
        m = jnp.min(d, axis=1, keepdims=True)
        j = jnp.min(jnp.where(d == m, iota, n), axis=1, keepdims=True)
        out = jnp.where(colk == i, jnp.broadcast_to(j, (qn, k)), out)
        d = jnp.where(jnp.broadcast_to(j, (qn, n)) == iota, jnp.inf, d)
        return d, out

    o_ref[0] = jax.lax.fori_loop(0, k, round_, (d, out))[1]


def _knn(k, xyz):
    b, n, _ = xyz.shape
    xt = jnp.transpose(xyz, (0, 2, 1))
    qb = min(n, 256)
    return pl.pallas_call(
        lambda q_ref, xt_ref, o_ref: _knn_body(q_ref, xt_ref, o_ref, k=k),
        grid=(b, n // qb),
        in_specs=[pl.BlockSpec((1, qb, 3), lambda i, j: (i, j, 0)),
                  pl.BlockSpec((1, 3, n), lambda i, j: (i, 0, 0))],
        out_specs=pl.BlockSpec((1, qb, k), lambda i, j: (i, j, 0)),
        out_shape=jax.ShapeDtypeStruct((b, n, k), jnp.int32),
    )(xyz, xt)


def _ball_body(q_ref, xt_ref, o_ref, *, nsample, r2):
    qn, n = q_ref.shape[1], xt_ref.shape[2]
    d = _pair_dists(q_ref, xt_ref)
    iota = jax.lax.broadcasted_iota(jnp.int32, (qn, n), 1)
    colk = jax.lax.broadcasted_iota(jnp.int32, (qn, nsample), 1)
    key = jnp.where(d > r2, n, iota)
    out = _concrete_zero_i32((qn, nsample))

    def round_(i, carry):
        key, out = carry
        j = jnp.min(key, axis=1, keepdims=True)
        out = jnp.where(colk == i, jnp.broadcast_to(j, (qn, nsample)), out)
        key = jnp.where(jnp.broadcast_to(j, (qn, n)) == iota, n, key)
        return key, out

    out = jax.lax.fori_loop(0, nsample, round_, (key, out))[1]
    first = jnp.broadcast_to(out[:, 0:1], (qn, nsample))
    o_ref[0] = jnp.where(out == n, first, out)


def _ball_query(radius, nsample, xyz, new_xyz):
    b, n, _ = xyz.shape
    s = new_xyz.shape[1]
    xt = jnp.transpose(xyz, (0, 2, 1))
    qb = min(s, 512)
    return pl.pallas_call(
        lambda q_ref, xt_ref, o_ref: _ball_body(
            q_ref, xt_ref, o_ref, nsample=nsample, r2=radius ** 2),
        grid=(b, s // qb),
        in_specs=[pl.BlockSpec((1, qb, 3), lambda i, j: (i, j, 0)),
                  pl.BlockSpec((1, 3, n), lambda i, j: (i, 0, 0))],
        out_specs=pl.BlockSpec((1, qb, nsample), lambda i, j: (i, j, 0)),
        out_shape=jax.ShapeDtypeStruct((b, s, nsample), jnp.int32),
    )(new_xyz, xt)


def _umbrella_surface(center, p, k):
    b, n, _ = center.shape
    idx = _knn(k, center)
    gxyz = _gather_rows(center, idx[:, :, 1:])
    gnorm = gxyz - center[:, :, None, :]
    phi = _xyz2sphere(gnorm)[..., 2]
    kk1 = k - 1
    ar = jnp.arange(kk1)
    lt = phi[..., None, :] < phi[..., :, None]
    eq = (phi[..., None, :] == phi[..., :, None]) & (ar[None, :] < ar[:, None])
    rank = jnp.sum(lt | eq, axis=-1)
    perm = (rank[..., None, :] == ar[:, None]).astype(_F32)
    sg = jnp.matmul(perm, gnorm)[:, :, :, None, :]
    sgr = jnp.roll(sg, -1, axis=2)
    g = jnp.concatenate([jnp.zeros_like(sg), sg, sgr], axis=-2)

    e1 = g[..., 1, :] - g[..., 0, :]
    e2 = g[..., 2, :] - g[..., 0, :]
    nor = jnp.cross(e1, e2)
    unit = nor / jnp.maximum(
        jnp.linalg.norm(nor, axis=-1, keepdims=True), _EPS)
    pos_mask = (unit[..., 0:1, 0] > 0).astype(_F32) * 2.0 - 1.0
    unit = unit * pos_mask[..., None]
    gc = jnp.mean(g, axis=-2)
    gpolar = _xyz2sphere(gc)
    gpos = jnp.sum(unit * gc, axis=-1, keepdims=True) / np.sqrt(3.0)
    feat = jnp.concatenate([gc, gpolar, unit, gpos], axis=-1)
    feat = jnp.nan_to_num(feat)

    kk = k - 1
    rows = b * n * kk
    x = feat.reshape(rows, feat.shape[-1])
    y1, s1 = _mm_stats(x, p['W1'].T, p['b1'])
    a1, c1 = _bn_affine(s1, rows, p['g1'], p['bt1'])
    out = _sum_mm(y1, p['W2'].T, p['b2'] * kk, (a1, c1), kk)
    return out.reshape(b, n, -1)


def _sa_stage(center, normal, feature, p, npoint, radius, nsample,
              pos_channel, group_all):
    b = center.shape[0]
    if group_all:
        new_center = jnp.zeros((b, 1, 3), dtype=center.dtype)
        new_normal = jnp.zeros((b, 1, normal.shape[-1]), dtype=normal.dtype)
        parts = [center[:, None], normal[:, None]]
        if feature is not None:
            parts.append(feature[:, None])
        nf = jnp.concatenate(parts, axis=-1)
        k = center.shape[1]
        s = nf.shape[1]
        rows = b * s * k
        cin = nf.shape[-1]
        flat = nf.reshape(rows, cin)
        xp = flat[:, :pos_channel]
        xf = flat[:, pos_channel:]
    else:
        parts = [center, normal]
        if feature is not None:
            parts.append(feature)
        table = jnp.concatenate(parts, axis=-1)
        fps_idx = _fps(center, npoint)
        sampled = _gather_rows(table, fps_idx)
        new_center = sampled[..., :3]
        new_normal = sampled[..., 3:3 + normal.shape[-1]]
        idx = _ball_query(radius, nsample, center, new_center)
        nbr = _gather_rows(table, idx)
        k = nsample
        s = npoint
        rows = b * s * k
        cin = table.shape[-1]
        flat = nbr.reshape(rows, cin)
        xp = (flat[:, :pos_channel]
              - jnp.broadcast_to(new_center[:, :, None, :],
                                 (b, s, k, 3)).reshape(rows, 3))
        xf = flat[:, pos_channel:]

    wlt, wft = p['Wl0'].T, p['Wf0'].T
    sl, sf = _l0_stats(xp, xf, wlt, p['bl0'], wft, p['bf0'])
    aff_l = _bn_affine(sl, rows, p['gl0'], p['btl0'])
    aff_f = _bn_affine(sf, rows, p['gf0'], p['btf0'])

    lays = p['layers']
    y, st = _l0_mm2(xp, xf, wlt, p['bl0'], wft, p['bf0'],
                    aff_l, aff_f, lays[0]['W'].T, lays[0]['b'])
    aff = _bn_affine(st, rows, lays[0]['g'], lays[0]['bt'])
    for lay in lays[1:-1]:
        y, st = _mm_stats(y, lay['W'].T, lay['b'], affine=aff)
        aff = _bn_affine(st, rows, lay['g'], lay['bt'])
    y, st = _mm_pool_stats(y, lays[-1]['W'].T, lays[-1]['b'], aff, k)
    aff = _bn_affine(st, rows, lays[-1]['g'], lays[-1]['bt'])
    f = _affine_relu(y, aff[0], aff[1])
    return new_center, new_normal, f.reshape(b, s, -1)


def kernel(points, params):
    center = jnp.transpose(points[:, :3, :], (0, 2, 1))
    normal = _umbrella_surface(center, params['umb'], 9)
    c, n, f = _sa_stage(center, normal, None, params['sa1'],
                        512, 0.1, 24, 3, False)
    c, n, f = _sa_stage(c, n, f, params['sa2'], 128, 0.2, 24, 3, False)
    c, n, f = _sa_stage(c, n, f, params['sa3'], 32, 0.4, 24, 3, False)
    c, n, f = _sa_stage(c, n, f, params['sa4'], None, None, None, 3, True)
    feat = f.reshape(-1, 2048)
    return _classifier(feat, params['cls'])
